# Initial kernel scaffold; baseline (speedup 1.0000x reference)
import functools
import numpy as np
import jax
import jax.numpy as jnp
from jax import lax
from jax.experimental import pallas as pl
from jax.experimental.pallas import tpu as pltpu
from jax.experimental.pallas import tpu_sc as plsc

B, N, K = 4, 512, 30
HID = 128
NH, DH = 4, 32
NPE, NRBF = 16, 16
VOCAB = 20
SLAB = 256
NSLAB = N // SLAB

_PREC = lax.Precision.HIGHEST


def _dot(a, b):
    return lax.dot_general(a, b, (((1,), (0,)), ((), ())), precision=_PREC,
                           preferred_element_type=jnp.float32)


def _ln_rows(x, g, b):
    n = x.shape[-1]
    mu = jnp.mean(x, -1, keepdims=True)
    d = x - mu
    var = jnp.sum(d * d, -1, keepdims=True) / (n - 1)
    sigma = jnp.sqrt(var + 1e-6)
    return g * d / (sigma + 1e-6) + b


def _normalize3(v, eps=1e-12):
    n = jnp.sqrt(jnp.sum(v * v, -1, keepdims=True))
    return v / jnp.maximum(n, eps)


def _cross(a, b):
    ax, ay, az = a[:, 0:1], a[:, 1:2], a[:, 2:3]
    bx, by, bz = b[:, 0:1], b[:, 1:2], b[:, 2:3]
    return jnp.concatenate([ay * bz - az * by, az * bx - ax * bz,
                            ax * by - ay * bx], 1)


def _dot3(a, b):
    return jnp.sum(a * b, -1, keepdims=True)


def _shift_up(z):
    return jnp.concatenate([z[1:], jnp.zeros((1, z.shape[1]), z.dtype)], 0)


def _shift_down(z, fill=0.0):
    return jnp.concatenate([jnp.full((1, z.shape[1]), fill, z.dtype), z[:-1]], 0)


def _quaternion_cols(R):
    Rxx, Ryy, Rzz = R[0], R[4], R[8]
    m1 = 0.5 * jnp.sqrt(jnp.abs(1.0 + Rxx - Ryy - Rzz) + 1e-10)
    m2 = 0.5 * jnp.sqrt(jnp.abs(1.0 - Rxx + Ryy - Rzz) + 1e-10)
    m3 = 0.5 * jnp.sqrt(jnp.abs(1.0 - Rxx - Ryy + Rzz) + 1e-10)
    s1 = jnp.sign(R[7] - R[5])
    s2 = jnp.sign(R[2] - R[6])
    s3 = jnp.sign(R[3] - R[1])
    w = jnp.sqrt(jax.nn.relu(1.0 + Rxx + Ryy + Rzz) + 1e-10) / 2.0
    q = jnp.concatenate([s1 * m1, s2 * m2, s3 * m3, w], 1)
    return _normalize3(q)


def _feature_body(x_ref, xcat_ref, node_w, node_b, nn_g, nn_b, edge_w, edge_b,
                  ne_g, ne_b, w_v, b_v, w_e, b_e, wkv0,
                  he_ref, hv_ref, eidx_ref, idxg_ref, t_ref):
    x = x_ref[0]
    xcat = xcat_ref[0]
    xca = x[:, 3:6]

    D2 = jnp.zeros((N, N), jnp.float32)
    for c in range(3):
        dc = xca[:, c:c + 1] - xcat[c:c + 1, :]
        D2 = D2 + dc * dc
    D = jnp.sqrt(D2 + 1e-6)

    iota_l = lax.broadcasted_iota(jnp.int32, (N, N), 1)
    Dw = D
    d_cols, i_cols = [], []
    for _ in range(K):
        m = jnp.min(Dw, axis=1, keepdims=True)
        sel = jnp.min(jnp.where(Dw == m, iota_l, N), axis=1, keepdims=True)
        d_cols.append(m)
        i_cols.append(sel)
        Dw = jnp.where(iota_l == sel, jnp.float32(np.inf), Dw)

    nA, cA, cC = x[:, 0:3], x[:, 3:6], x[:, 6:9]
    vA = cA - nA
    vB = cC - cA
    nA1 = _shift_up(nA)
    vC = nA1 - cC
    uA = _normalize3(vA)
    uB = _normalize3(vB)
    uC = _normalize3(vC)
    uA1 = _shift_up(uA)
    uB1 = _shift_up(uB)

    def dihed_cs(u2, u1, u0):
        n2 = _normalize3(_cross(u2, u1))
        n1 = _normalize3(_cross(u1, u0))
        cosD = jnp.clip(_dot3(n2, n1), -1.0 + 1e-7, 1.0 - 1e-7)
        sinD = jnp.sign(_dot3(u2, n1)) * jnp.sqrt(1.0 - cosD * cosD)
        return cosD, sinD

    rows = lax.broadcasted_iota(jnp.int32, (N, 1), 0)
    c1, s1 = dihed_cs(uA, uB, uC)
    c2, s2 = dihed_cs(uB, uC, uA1)
    c3, s3 = dihed_cs(uC, uA1, uB1)
    ok = rows <= N - 2
    c1 = jnp.where(ok, c1, 1.0)
    s1 = jnp.where(ok, s1, 0.0)
    c2 = jnp.where(ok, c2, 1.0)
    s2 = jnp.where(ok, s2, 0.0)
    c0 = _shift_down(c3, 1.0)
    s0 = _shift_down(s3, 0.0)
    Vf = jnp.concatenate([c0, c1, c2, s0, s1, s2], 1)

    xca1 = _shift_up(xca)
    Uc = _normalize3(xca1 - xca)
    Uc1 = _shift_up(Uc)
    o1 = _normalize3(Uc - Uc1)
    n2v = _normalize3(_cross(Uc, Uc1))
    o3 = _cross(o1, n2v)
    om9_raw = jnp.concatenate([o1, n2v, o3], 1)
    om9_sh = _shift_down(om9_raw, 0.0)
    okr = jnp.logical_and(rows >= 1, rows <= N - 3)
    Om9 = jnp.where(okr, om9_sh, 0.0)

    TBL = jnp.concatenate([xca, Om9], 1)

    freq = np.exp(np.arange(0, NPE, 2, dtype=np.float32) * (-np.log(10000.0) / NPE))
    freq = jnp.asarray(freq.reshape(1, NPE // 2))
    mu = jnp.asarray(np.linspace(0.0, 20.0, NRBF, dtype=np.float32).reshape(1, NRBF))
    inv_sig = jnp.float32(NRBF / 20.0)
    n_f = rows.astype(jnp.float32)

    he_blocks = []
    for k in range(K):
        sel = i_cols[k]
        dk = sel.astype(jnp.float32) - n_f
        ang = dk * freq
        epos = jnp.concatenate([jnp.cos(ang), jnp.sin(ang)], 1)
        dd = d_cols[k]
        t = (dd - mu) * inv_sig
        rbf = jnp.exp(-t * t)

        oh = (iota_l == sel).astype(jnp.float32)
        Gk = _dot(oh, TBL)
        xn = Gk[:, 0:3]
        on9 = Gk[:, 3:12]
        dXn = xn - xca
        du_cols = []
        for i in range(3):
            acc = (Om9[:, 3 * i:3 * i + 1] * dXn[:, 0:1]
                   + Om9[:, 3 * i + 1:3 * i + 2] * dXn[:, 1:2]
                   + Om9[:, 3 * i + 2:3 * i + 3] * dXn[:, 2:3])
            du_cols.append(acc)
        du = _normalize3(jnp.concatenate(du_cols, 1))
        Rcols = []
        for i in range(3):
            for l in range(3):
                r = (Om9[:, 0 + i:1 + i] * on9[:, 0 + l:1 + l]
                     + Om9[:, 3 + i:4 + i] * on9[:, 3 + l:4 + l]
                     + Om9[:, 6 + i:7 + i] * on9[:, 6 + l:7 + l])
                Rcols.append(r)
        q = _quaternion_cols(Rcols)
        of_k = jnp.concatenate([du, q], 1)

        e_k = jnp.concatenate([epos, rbf, of_k], 1)
        he = _ln_rows(_dot(e_k, edge_w[...]) + edge_b[...], ne_g[...], ne_b[...])
        he_blocks.append(_dot(he, w_e[...]) + b_e[...])

    he_ref[0] = jnp.concatenate(he_blocks, 1)

    v = _ln_rows(_dot(Vf, node_w[...]) + node_b[...], nn_g[...], nn_b[...])
    hv = _dot(v, w_v[...]) + b_v[...]
    hv_ref[0] = hv
    t_ref[0] = _dot(hv, wkv0[...])

    eidx = jnp.concatenate(i_cols, 1)
    eidx_ref[0] = eidx
    idxg_ref[0] = eidx + pl.program_id(0) * N


def _layer_body(is_dec, next_kind, C, *refs):
    it = iter(refs)
    hv_ref = next(it)
    he_ref = next(it)
    g_ref = next(it)
    eidx_ref = next(it) if is_dec else None
    if next_kind == 'dec':
        s_ref = next(it)
        w_s = next(it)
        hve_ref = next(it) if is_dec else None
    wq = next(it)
    wkv_e = next(it)
    wo = next(it)
    n0g = next(it)
    n0b = next(it)
    wi = next(it)
    bi = next(it)
    wo2 = next(it)
    bo = next(it)
    n1g = next(it)
    n1b = next(it)
    if next_kind == 'enc':
        wnext = next(it)
    elif next_kind == 'dec':
        wnA = next(it)
        wnB = next(it)
    else:
        wout = next(it)
        bout = next(it)
    hv_out = next(it)
    t_out = next(it)

    hv = hv_ref[0]
    he = he_ref[0]
    g = g_ref[0]

    q = _dot(hv, wq[...])
    scale = jnp.float32(1.0 / np.sqrt(DH))
    if is_dec:
        srow = pl.program_id(1) * SLAB
        n_col = lax.broadcasted_iota(jnp.int32, (SLAB, 1), 0) + srow
        eidx = eidx_ref[0]

    logit_cols = [[] for _ in range(NH)]
    vbuf = []
    for k in range(K):
        ek = he[:, HID * k:HID * (k + 1)]
        kv = _dot(ek, wkv_e[...])
        kcol = kv[:, :HID]
        vcol = kv[:, HID:]
        base = C * k
        if not is_dec:
            kk = kcol + g[:, base:base + HID]
            vv = vcol + g[:, base + HID:base + 2 * HID]
        else:
            bw = (eidx[:, k:k + 1] < n_col).astype(jnp.float32)
            fw = 1.0 - bw
            kk = (kcol + bw * g[:, base:base + HID]
                  + fw * g[:, base + 2 * HID:base + 3 * HID])
            vv = (vcol + bw * g[:, base + HID:base + 2 * HID]
                  + fw * g[:, base + 3 * HID:base + 4 * HID])
        vbuf.append(vv)
        for h in range(NH):
            sl = slice(DH * h, DH * (h + 1))
            lh = jnp.sum(q[:, sl] * kk[:, sl], axis=1, keepdims=True)
            logit_cols[h].append(lh * scale)

    accs = []
    for h in range(NH):
        lg = jnp.concatenate(logit_cols[h], 1)
        m = jnp.max(lg, 1, keepdims=True)
        e = jnp.exp(lg - m)
        a = e / jnp.sum(e, 1, keepdims=True)
        acc = jnp.zeros((SLAB, DH), jnp.float32)
        for k in range(K):
            acc = acc + a[:, k:k + 1] * vbuf[k][:, DH * h:DH * (h + 1)]
        accs.append(acc)
    upd = _dot(jnp.concatenate(accs, 1), wo[...])

    h1 = _ln_rows(hv + upd, n0g[...], n0b[...])
    ffn = _dot(jax.nn.relu(_dot(h1, wi[...]) + bi[...]), wo2[...]) + bo[...]
    h2 = _ln_rows(h1 + ffn, n1g[...], n1b[...])
    hv_out[0] = h2

    if next_kind == 'enc':
        t_out[0] = _dot(h2, wnext[...])
    elif next_kind == 'dec':
        s_col = s_ref[0]
        iota20 = lax.broadcasted_iota(jnp.int32, (1, VOCAB), 1)
        oh_s = (s_col == iota20).astype(jnp.float32)
        hs = _dot(oh_s, w_s[...])
        hve = h2 if not is_dec else hve_ref[0]
        t_out[0] = jnp.concatenate(
            [_dot(hs, wnA[...]) + _dot(h2, wnB[...]), _dot(hve, wnB[...])], 1)
    else:
        lg = _dot(h2, wout[...]) + bout[...]
        m = jnp.max(lg, 1, keepdims=True)
        t_out[0] = lg - m - jnp.log(jnp.sum(jnp.exp(lg - m), 1, keepdims=True))


def _full_spec(shape):
    nd = len(shape)
    return pl.BlockSpec(shape, lambda b, s, _n=nd: (0,) * _n)


def _slab_spec(f):
    return pl.BlockSpec((1, SLAB, f), lambda b, s: (b, s, 0))


def _feature_call(Xr, XcaT, fp, w_v, b_v, w_e, b_e, wkv0):
    in_specs = [pl.BlockSpec((1, N, 12), lambda b: (b, 0, 0)),
                pl.BlockSpec((1, 3, N), lambda b: (b, 0, 0))]
    weights = [fp['node_W'], fp['node_b'].reshape(1, -1), fp['nn_g'].reshape(1, -1),
               fp['nn_b'].reshape(1, -1), fp['edge_W'], fp['edge_b'].reshape(1, -1),
               fp['ne_g'].reshape(1, -1), fp['ne_b'].reshape(1, -1),
               w_v, b_v.reshape(1, -1), w_e, b_e.reshape(1, -1), wkv0]
    for w in weights:
        in_specs.append(pl.BlockSpec(w.shape, lambda b, _n=len(w.shape): (0,) * _n))
    out_shape = [jax.ShapeDtypeStruct((B, N, K * HID), jnp.float32),
                 jax.ShapeDtypeStruct((B, N, HID), jnp.float32),
                 jax.ShapeDtypeStruct((B, N, K), jnp.int32),
                 jax.ShapeDtypeStruct((B, N, K), jnp.int32),
                 jax.ShapeDtypeStruct((B, N, 2 * HID), jnp.float32)]
    out_specs = [pl.BlockSpec((1, N, K * HID), lambda b: (b, 0, 0)),
                 pl.BlockSpec((1, N, HID), lambda b: (b, 0, 0)),
                 pl.BlockSpec((1, N, K), lambda b: (b, 0, 0)),
                 pl.BlockSpec((1, N, K), lambda b: (b, 0, 0)),
                 pl.BlockSpec((1, N, 2 * HID), lambda b: (b, 0, 0))]
    return pl.pallas_call(
        _feature_body, grid=(B,), in_specs=in_specs, out_specs=out_specs,
        out_shape=out_shape)(Xr, XcaT, *weights)


def _layer_call(is_dec, next_kind, C, hv, he, g, eidx, s_col, w_s, hve, lp,
                next_w):
    args = [hv, he, g]
    in_specs = [_slab_spec(HID), _slab_spec(K * HID), _slab_spec(K * C)]
    if is_dec:
        args.append(eidx)
        in_specs.append(_slab_spec(K))
    if next_kind == 'dec':
        args.append(s_col)
        in_specs.append(_slab_spec(1))
        args.append(w_s)
        in_specs.append(_full_spec(w_s.shape))
        if is_dec:
            args.append(hve)
            in_specs.append(_slab_spec(HID))
    weights = [lp['WQ'], jnp.concatenate([lp['WK'][:HID], lp['WV'][:HID]], 1),
               lp['WO'], lp['n0_g'].reshape(1, -1), lp['n0_b'].reshape(1, -1),
               lp['Wi'], lp['bi'].reshape(1, -1), lp['Wo'],
               lp['bo'].reshape(1, -1), lp['n1_g'].reshape(1, -1),
               lp['n1_b'].reshape(1, -1)]
    weights += [w for w in next_w]
    for w in weights:
        args.append(w)
        in_specs.append(_full_spec(w.shape))

    out_shape = [jax.ShapeDtypeStruct((B, N, HID), jnp.float32)]
    out_specs = [_slab_spec(HID)]
    if next_kind == 'enc':
        out_shape.append(jax.ShapeDtypeStruct((B, N, 2 * HID), jnp.float32))
        out_specs.append(_slab_spec(2 * HID))
    elif next_kind == 'dec':
        out_shape.append(jax.ShapeDtypeStruct((B, N, 4 * HID), jnp.float32))
        out_specs.append(_slab_spec(4 * HID))
    else:
        out_shape.append(jax.ShapeDtypeStruct((B, N, VOCAB), jnp.float32))
        out_specs.append(_slab_spec(VOCAB))

    body = functools.partial(_layer_body, is_dec, next_kind, C)
    return pl.pallas_call(
        body, grid=(B, NSLAB), in_specs=in_specs, out_specs=out_specs,
        out_shape=out_shape)(*args)


def _sc_gather(table, idx, C):
    M = idx.shape[0]
    NW = 32
    per_w = M // NW
    chunk = 384 if C <= 256 else 192
    n_it = per_w // chunk
    mesh = plsc.VectorSubcoreMesh(core_axis_name="c", subcore_axis_name="s")

    def body(table_ref, idx_ref, out_ref, idx_v, rows_v, sem):
        wid = lax.axis_index("s") * 2 + lax.axis_index("c")
        base = wid * per_w

        def it(i, carry):
            off = base + i * chunk
            pltpu.sync_copy(idx_ref.at[pl.ds(off, chunk)], idx_v)
            pltpu.async_copy(table_ref.at[idx_v], rows_v, sem).wait()
            pltpu.sync_copy(rows_v, out_ref.at[pl.ds(off, chunk)])
            return carry

        lax.fori_loop(0, n_it, it, 0)

    f = pl.kernel(body,
                  out_type=jax.ShapeDtypeStruct((M, C), jnp.float32),
                  mesh=mesh,
                  scratch_types=[pltpu.VMEM((chunk,), jnp.int32),
                                 pltpu.VMEM((chunk, C), jnp.float32),
                                 pltpu.SemaphoreType.DMA])
    return f(table, idx)


_gather = _sc_gather


def kernel(X, S, L, mask, params):
    fp = params['feat']
    enc = params['enc']
    dec = params['dec']
    Xr = X.reshape(B, N, 12)
    XcaT = jnp.transpose(X[:, :, 1, :], (0, 2, 1))
    s_col = S.reshape(B, N, 1).astype(jnp.int32)

    def enc_tbl_w(lp):
        return jnp.concatenate([lp['WK'][HID:], lp['WV'][HID:]], 1)

    def dec_tbl_w(lp):
        wnA = jnp.concatenate([lp['WK'][HID:2 * HID], lp['WV'][HID:2 * HID]], 1)
        wnB = jnp.concatenate([lp['WK'][2 * HID:], lp['WV'][2 * HID:]], 1)
        return wnA, wnB

    hE, hV, eidx, idxg, T = _feature_call(
        Xr, XcaT, fp, params['W_v'], params['b_v'], params['W_e'],
        params['b_e'], enc_tbl_w(enc[0]))
    idx_flat = idxg.reshape(B * N * K)

    for i in range(3):
        lp = enc[i]
        G = _gather(T.reshape(B * N, 2 * HID), idx_flat, 2 * HID)
        G = G.reshape(B, N, K * 2 * HID)
        if i < 2:
            nk, nw = 'enc', (enc_tbl_w(enc[i + 1]),)
        else:
            nk, nw = 'dec', dec_tbl_w(dec[0])
        hV, T = _layer_call(False, nk, 2 * HID, hV, hE, G, None,
                            s_col if nk == 'dec' else None,
                            params['W_s'] if nk == 'dec' else None,
                            None, lp, nw)

    hVe = hV
    for i in range(3):
        lp = dec[i]
        G = _gather(T.reshape(B * N, 4 * HID), idx_flat, 4 * HID)
        G = G.reshape(B, N, K * 4 * HID)
        if i < 2:
            nk, nw = 'dec', dec_tbl_w(dec[i + 1])
        else:
            nk, nw = None, (params['W_out'], params['b_out'].reshape(1, -1))
        hV, T = _layer_call(True, nk, 4 * HID, hV, hE, G, eidx,
                            s_col if nk == 'dec' else None,
                            params['W_s'] if nk == 'dec' else None,
                            hVe if nk == 'dec' else None, lp, nw)
    return T


# speedup vs baseline: 2.6675x; 2.6675x over previous
import functools
import numpy as np
import jax
import jax.numpy as jnp
from jax import lax
from jax.experimental import pallas as pl
from jax.experimental.pallas import tpu as pltpu
from jax.experimental.pallas import tpu_sc as plsc

B, N, K = 4, 512, 30
HID = 128
NH, DH = 4, 32
NPE, NRBF = 16, 16
VOCAB = 20
SLAB = 128
NSLAB = N // SLAB

_PREC = lax.Precision.HIGHEST


def _dot(a, b):
    return lax.dot_general(a, b, (((1,), (0,)), ((), ())), precision=_PREC,
                           preferred_element_type=jnp.float32)


def _ln_rows(x, g, b):
    n = x.shape[-1]
    mu = jnp.mean(x, -1, keepdims=True)
    d = x - mu
    var = jnp.sum(d * d, -1, keepdims=True) / (n - 1)
    sigma = jnp.sqrt(var + 1e-6)
    return g * d / (sigma + 1e-6) + b


def _normalize3(v, eps=1e-12):
    n = jnp.sqrt(jnp.sum(v * v, -1, keepdims=True))
    return v / jnp.maximum(n, eps)


def _cross(a, b):
    ax, ay, az = a[:, 0:1], a[:, 1:2], a[:, 2:3]
    bx, by, bz = b[:, 0:1], b[:, 1:2], b[:, 2:3]
    return jnp.concatenate([ay * bz - az * by, az * bx - ax * bz,
                            ax * by - ay * bx], 1)


def _dot3(a, b):
    return jnp.sum(a * b, -1, keepdims=True)


def _shift_up(z):
    return jnp.concatenate([z[1:], jnp.zeros((1, z.shape[1]), z.dtype)], 0)


def _shift_down(z, fill=0.0):
    return jnp.concatenate([jnp.full((1, z.shape[1]), fill, z.dtype), z[:-1]], 0)


def _quaternion_cols(R):
    Rxx, Ryy, Rzz = R[0], R[4], R[8]
    m1 = 0.5 * jnp.sqrt(jnp.abs(1.0 + Rxx - Ryy - Rzz) + 1e-10)
    m2 = 0.5 * jnp.sqrt(jnp.abs(1.0 - Rxx + Ryy - Rzz) + 1e-10)
    m3 = 0.5 * jnp.sqrt(jnp.abs(1.0 - Rxx - Ryy + Rzz) + 1e-10)
    s1 = jnp.sign(R[7] - R[5])
    s2 = jnp.sign(R[2] - R[6])
    s3 = jnp.sign(R[3] - R[1])
    w = jnp.sqrt(jax.nn.relu(1.0 + Rxx + Ryy + Rzz) + 1e-10) / 2.0
    q = jnp.concatenate([s1 * m1, s2 * m2, s3 * m3, w], 1)
    return _normalize3(q)


def _feature_body(x_ref, xcat_ref, node_w, node_b, nn_g, nn_b, edge_w, edge_b,
                  ne_g, ne_b, w_v, b_v, w_e, b_e, wkv0,
                  he_ref, hv_ref, eidx_ref, idxg_ref, t_ref):
    x = x_ref[0]
    xcat = xcat_ref[0]
    xca = x[:, 3:6]

    D2 = jnp.zeros((N, N), jnp.float32)
    for c in range(3):
        dc = xca[:, c:c + 1] - xcat[c:c + 1, :]
        D2 = D2 + dc * dc
    D = jnp.sqrt(D2 + 1e-6)

    iota_l = lax.broadcasted_iota(jnp.int32, (N, N), 1)
    iota_k = lax.broadcasted_iota(jnp.int32, (N, K), 1)
    Dw = D
    Dn = jnp.zeros((N, K), jnp.float32)
    Ei = jnp.zeros((N, K), jnp.int32)
    for kk in range(K):
        m = jnp.min(Dw, axis=1, keepdims=True)
        sel = jnp.min(jnp.where(Dw == m, iota_l, N), axis=1, keepdims=True)
        Dn = jnp.where(iota_k == kk, m, Dn)
        Ei = jnp.where(iota_k == kk, sel, Ei)
        Dw = jnp.where(iota_l == sel, jnp.float32(np.inf), Dw)
    eidx_ref[0] = Ei
    idxg_ref[0] = Ei + pl.program_id(0) * N

    nA, cA, cC = x[:, 0:3], x[:, 3:6], x[:, 6:9]
    vA = cA - nA
    vB = cC - cA
    nA1 = _shift_up(nA)
    vC = nA1 - cC
    uA = _normalize3(vA)
    uB = _normalize3(vB)
    uC = _normalize3(vC)
    uA1 = _shift_up(uA)
    uB1 = _shift_up(uB)

    def dihed_cs(u2, u1, u0):
        n2 = _normalize3(_cross(u2, u1))
        n1 = _normalize3(_cross(u1, u0))
        cosD = jnp.clip(_dot3(n2, n1), -1.0 + 1e-7, 1.0 - 1e-7)
        sinD = jnp.sign(_dot3(u2, n1)) * jnp.sqrt(1.0 - cosD * cosD)
        return cosD, sinD

    rows = lax.broadcasted_iota(jnp.int32, (N, 1), 0)
    c1, s1 = dihed_cs(uA, uB, uC)
    c2, s2 = dihed_cs(uB, uC, uA1)
    c3, s3 = dihed_cs(uC, uA1, uB1)
    ok = rows <= N - 2
    c1 = jnp.where(ok, c1, 1.0)
    s1 = jnp.where(ok, s1, 0.0)
    c2 = jnp.where(ok, c2, 1.0)
    s2 = jnp.where(ok, s2, 0.0)
    c0 = _shift_down(c3, 1.0)
    s0 = _shift_down(s3, 0.0)
    Vf = jnp.concatenate([c0, c1, c2, s0, s1, s2], 1)

    xca1 = _shift_up(xca)
    Uc = _normalize3(xca1 - xca)
    Uc1 = _shift_up(Uc)
    o1 = _normalize3(Uc - Uc1)
    n2v = _normalize3(_cross(Uc, Uc1))
    o3 = _cross(o1, n2v)
    om9_raw = jnp.concatenate([o1, n2v, o3], 1)
    om9_sh = _shift_down(om9_raw, 0.0)
    okr = jnp.logical_and(rows >= 1, rows <= N - 3)
    Om9 = jnp.where(okr, om9_sh, 0.0)

    TBL = jnp.concatenate([xca, Om9], 1)

    iota8 = lax.broadcasted_iota(jnp.int32, (1, NPE // 2), 1).astype(jnp.float32)
    freq = jnp.exp(iota8 * jnp.float32(-2.0 * np.log(10000.0) / NPE))
    iota16 = lax.broadcasted_iota(jnp.int32, (1, NRBF), 1).astype(jnp.float32)
    mu = iota16 * jnp.float32(20.0 / (NRBF - 1))
    inv_sig = jnp.float32(NRBF / 20.0)
    n_f = rows.astype(jnp.float32)

    for k in range(K):
        sel = Ei[:, k:k + 1]
        dk = sel.astype(jnp.float32) - n_f
        ang = dk * freq
        epos = jnp.concatenate([jnp.cos(ang), jnp.sin(ang)], 1)
        dd = Dn[:, k:k + 1]
        t = (dd - mu) * inv_sig
        rbf = jnp.exp(-t * t)

        oh = (iota_l == sel).astype(jnp.float32)
        Gk = _dot(oh, TBL)
        xn = Gk[:, 0:3]
        on9 = Gk[:, 3:12]
        dXn = xn - xca
        du_cols = []
        for i in range(3):
            acc = (Om9[:, 3 * i:3 * i + 1] * dXn[:, 0:1]
                   + Om9[:, 3 * i + 1:3 * i + 2] * dXn[:, 1:2]
                   + Om9[:, 3 * i + 2:3 * i + 3] * dXn[:, 2:3])
            du_cols.append(acc)
        du = _normalize3(jnp.concatenate(du_cols, 1))
        Rcols = []
        for i in range(3):
            for l in range(3):
                r = (Om9[:, 0 + i:1 + i] * on9[:, 0 + l:1 + l]
                     + Om9[:, 3 + i:4 + i] * on9[:, 3 + l:4 + l]
                     + Om9[:, 6 + i:7 + i] * on9[:, 6 + l:7 + l])
                Rcols.append(r)
        q = _quaternion_cols(Rcols)
        of_k = jnp.concatenate([du, q], 1)

        e_k = jnp.concatenate([epos, rbf, of_k], 1)
        he = _ln_rows(_dot(e_k, edge_w[...]) + edge_b[...], ne_g[...], ne_b[...])
        he_ref[0, :, HID * k:HID * (k + 1)] = _dot(he, w_e[...]) + b_e[...]

    v = _ln_rows(_dot(Vf, node_w[...]) + node_b[...], nn_g[...], nn_b[...])
    hv = _dot(v, w_v[...]) + b_v[...]
    hv_ref[0] = hv
    t_ref[0] = _dot(hv, wkv0[...])


def _layer_body(is_dec, next_kind, C, *refs):
    it = iter(refs)
    hv_ref = next(it)
    he_ref = next(it)
    g_ref = next(it)
    eidx_ref = next(it) if is_dec else None
    if next_kind == 'dec':
        s_ref = next(it)
        w_s = next(it)
        hve_ref = next(it) if is_dec else None
    wq = next(it)
    wkv_e = next(it)
    wo = next(it)
    n0g = next(it)
    n0b = next(it)
    wi = next(it)
    bi = next(it)
    wo2 = next(it)
    bo = next(it)
    n1g = next(it)
    n1b = next(it)
    if next_kind == 'enc':
        wnext = next(it)
    elif next_kind == 'dec':
        wnA = next(it)
        wnB = next(it)
    else:
        wout = next(it)
        bout = next(it)
    hv_out = next(it)
    t_out = next(it)

    hv = hv_ref[0]
    he = he_ref[0]
    g = g_ref[0]

    q = _dot(hv, wq[...])
    scale = jnp.float32(1.0 / np.sqrt(DH))
    if is_dec:
        srow = pl.program_id(1) * SLAB
        n_col = lax.broadcasted_iota(jnp.int32, (SLAB, 1), 0) + srow
        eidx = eidx_ref[0]

    iota_k = lax.broadcasted_iota(jnp.int32, (SLAB, K), 1)
    lgs = [jnp.zeros((SLAB, K), jnp.float32) for _ in range(NH)]
    vbuf = []
    for k in range(K):
        ek = he[:, HID * k:HID * (k + 1)]
        kv = _dot(ek, wkv_e[...])
        kcol = kv[:, :HID]
        vcol = kv[:, HID:]
        base = C * k
        if not is_dec:
            kk = kcol + g[:, base:base + HID]
            vv = vcol + g[:, base + HID:base + 2 * HID]
        else:
            bw = (eidx[:, k:k + 1] < n_col).astype(jnp.float32)
            fw = 1.0 - bw
            kk = (kcol + bw * g[:, base:base + HID]
                  + fw * g[:, base + 2 * HID:base + 3 * HID])
            vv = (vcol + bw * g[:, base + HID:base + 2 * HID]
                  + fw * g[:, base + 3 * HID:base + 4 * HID])
        vbuf.append(vv)
        for h in range(NH):
            sl = slice(DH * h, DH * (h + 1))
            lh = jnp.sum(q[:, sl] * kk[:, sl], axis=1, keepdims=True)
            lgs[h] = jnp.where(iota_k == k, lh * scale, lgs[h])

    accs = []
    for h in range(NH):
        lg = lgs[h]
        m = jnp.max(lg, 1, keepdims=True)
        e = jnp.exp(lg - m)
        a = e / jnp.sum(e, 1, keepdims=True)
        acc = jnp.zeros((SLAB, DH), jnp.float32)
        for k in range(K):
            acc = acc + a[:, k:k + 1] * vbuf[k][:, DH * h:DH * (h + 1)]
        accs.append(acc)
    upd = _dot(jnp.concatenate(accs, 1), wo[...])

    h1 = _ln_rows(hv + upd, n0g[...], n0b[...])
    ffn = _dot(jax.nn.relu(_dot(h1, wi[...]) + bi[...]), wo2[...]) + bo[...]
    h2 = _ln_rows(h1 + ffn, n1g[...], n1b[...])
    hv_out[0] = h2

    if next_kind == 'enc':
        t_out[0] = _dot(h2, wnext[...])
    elif next_kind == 'dec':
        s_col = s_ref[0]
        iota20 = lax.broadcasted_iota(jnp.int32, (1, VOCAB), 1)
        oh_s = (s_col == iota20).astype(jnp.float32)
        hs = _dot(oh_s, w_s[...])
        hve = h2 if not is_dec else hve_ref[0]
        t_out[0] = jnp.concatenate(
            [_dot(hs, wnA[...]) + _dot(h2, wnB[...]), _dot(hve, wnB[...])], 1)
    else:
        lg = _dot(h2, wout[...]) + bout[...]
        m = jnp.max(lg, 1, keepdims=True)
        t_out[0] = lg - m - jnp.log(jnp.sum(jnp.exp(lg - m), 1, keepdims=True))


def _full_spec(shape):
    nd = len(shape)
    return pl.BlockSpec(shape, lambda b, s, _n=nd: (0,) * _n)


def _slab_spec(f):
    return pl.BlockSpec((1, SLAB, f), lambda b, s: (b, s, 0))


def _feature_call(Xr, XcaT, fp, w_v, b_v, w_e, b_e, wkv0):
    in_specs = [pl.BlockSpec((1, N, 12), lambda b: (b, 0, 0)),
                pl.BlockSpec((1, 3, N), lambda b: (b, 0, 0))]
    weights = [fp['node_W'], fp['node_b'].reshape(1, -1), fp['nn_g'].reshape(1, -1),
               fp['nn_b'].reshape(1, -1), fp['edge_W'], fp['edge_b'].reshape(1, -1),
               fp['ne_g'].reshape(1, -1), fp['ne_b'].reshape(1, -1),
               w_v, b_v.reshape(1, -1), w_e, b_e.reshape(1, -1), wkv0]
    for w in weights:
        in_specs.append(pl.BlockSpec(w.shape, lambda b, _n=len(w.shape): (0,) * _n))
    out_shape = [jax.ShapeDtypeStruct((B, N, K * HID), jnp.float32),
                 jax.ShapeDtypeStruct((B, N, HID), jnp.float32),
                 jax.ShapeDtypeStruct((B, N, K), jnp.int32),
                 jax.ShapeDtypeStruct((B, N, K), jnp.int32),
                 jax.ShapeDtypeStruct((B, N, 2 * HID), jnp.float32)]
    out_specs = [pl.BlockSpec((1, N, K * HID), lambda b: (b, 0, 0)),
                 pl.BlockSpec((1, N, HID), lambda b: (b, 0, 0)),
                 pl.BlockSpec((1, N, K), lambda b: (b, 0, 0)),
                 pl.BlockSpec((1, N, K), lambda b: (b, 0, 0)),
                 pl.BlockSpec((1, N, 2 * HID), lambda b: (b, 0, 0))]
    return pl.pallas_call(
        _feature_body, grid=(B,), in_specs=in_specs, out_specs=out_specs,
        out_shape=out_shape)(Xr, XcaT, *weights)


def _layer_call(is_dec, next_kind, C, hv, he, g, eidx, s_col, w_s, hve, lp,
                next_w):
    args = [hv, he, g]
    in_specs = [_slab_spec(HID), _slab_spec(K * HID), _slab_spec(K * C)]
    if is_dec:
        args.append(eidx)
        in_specs.append(_slab_spec(K))
    if next_kind == 'dec':
        args.append(s_col)
        in_specs.append(_slab_spec(1))
        args.append(w_s)
        in_specs.append(_full_spec(w_s.shape))
        if is_dec:
            args.append(hve)
            in_specs.append(_slab_spec(HID))
    weights = [lp['WQ'], jnp.concatenate([lp['WK'][:HID], lp['WV'][:HID]], 1),
               lp['WO'], lp['n0_g'].reshape(1, -1), lp['n0_b'].reshape(1, -1),
               lp['Wi'], lp['bi'].reshape(1, -1), lp['Wo'],
               lp['bo'].reshape(1, -1), lp['n1_g'].reshape(1, -1),
               lp['n1_b'].reshape(1, -1)]
    weights += [w for w in next_w]
    for w in weights:
        args.append(w)
        in_specs.append(_full_spec(w.shape))

    out_shape = [jax.ShapeDtypeStruct((B, N, HID), jnp.float32)]
    out_specs = [_slab_spec(HID)]
    if next_kind == 'enc':
        out_shape.append(jax.ShapeDtypeStruct((B, N, 2 * HID), jnp.float32))
        out_specs.append(_slab_spec(2 * HID))
    elif next_kind == 'dec':
        out_shape.append(jax.ShapeDtypeStruct((B, N, 4 * HID), jnp.float32))
        out_specs.append(_slab_spec(4 * HID))
    else:
        out_shape.append(jax.ShapeDtypeStruct((B, N, VOCAB), jnp.float32))
        out_specs.append(_slab_spec(VOCAB))

    body = functools.partial(_layer_body, is_dec, next_kind, C)
    return pl.pallas_call(
        body, grid=(B, NSLAB), in_specs=in_specs, out_specs=out_specs,
        out_shape=out_shape)(*args)


def _sc_gather(table, idx, C):
    M = idx.shape[0]
    NW = 32
    per_w = M // NW
    chunk = 384 if C <= 256 else 192
    n_it = per_w // chunk
    mesh = plsc.VectorSubcoreMesh(core_axis_name="c", subcore_axis_name="s")

    def body(table_ref, idx_ref, out_ref, idx_v, rows_v, sem):
        wid = lax.axis_index("s") * 2 + lax.axis_index("c")
        base = wid * per_w

        def it(i, carry):
            off = base + i * chunk
            pltpu.sync_copy(idx_ref.at[pl.ds(off, chunk)], idx_v)
            pltpu.async_copy(table_ref.at[idx_v], rows_v, sem).wait()
            pltpu.sync_copy(rows_v, out_ref.at[pl.ds(off, chunk)])
            return carry

        lax.fori_loop(0, n_it, it, 0)

    f = pl.kernel(body,
                  out_type=jax.ShapeDtypeStruct((M, C), jnp.float32),
                  mesh=mesh,
                  scratch_types=[pltpu.VMEM((chunk,), jnp.int32),
                                 pltpu.VMEM((chunk, C), jnp.float32),
                                 pltpu.SemaphoreType.DMA])
    return f(table, idx)


_gather = _sc_gather


def kernel(X, S, L, mask, params):
    fp = params['feat']
    enc = params['enc']
    dec = params['dec']
    Xr = X.reshape(B, N, 12)
    XcaT = jnp.transpose(X[:, :, 1, :], (0, 2, 1))
    s_col = S.reshape(B, N, 1).astype(jnp.int32)

    def enc_tbl_w(lp):
        return jnp.concatenate([lp['WK'][HID:], lp['WV'][HID:]], 1)

    def dec_tbl_w(lp):
        wnA = jnp.concatenate([lp['WK'][HID:2 * HID], lp['WV'][HID:2 * HID]], 1)
        wnB = jnp.concatenate([lp['WK'][2 * HID:], lp['WV'][2 * HID:]], 1)
        return wnA, wnB

    hE, hV, eidx, idxg, T = _feature_call(
        Xr, XcaT, fp, params['W_v'], params['b_v'], params['W_e'],
        params['b_e'], enc_tbl_w(enc[0]))
    idx_flat = idxg.reshape(B * N * K)

    for i in range(3):
        lp = enc[i]
        G = _gather(T.reshape(B * N, 2 * HID), idx_flat, 2 * HID)
        G = G.reshape(B, N, K * 2 * HID)
        if i < 2:
            nk, nw = 'enc', (enc_tbl_w(enc[i + 1]),)
        else:
            nk, nw = 'dec', dec_tbl_w(dec[0])
        hV, T = _layer_call(False, nk, 2 * HID, hV, hE, G, None,
                            s_col if nk == 'dec' else None,
                            params['W_s'] if nk == 'dec' else None,
                            None, lp, nw)

    hVe = hV
    for i in range(3):
        lp = dec[i]
        G = _gather(T.reshape(B * N, 4 * HID), idx_flat, 4 * HID)
        G = G.reshape(B, N, K * 4 * HID)
        if i < 2:
            nk, nw = 'dec', dec_tbl_w(dec[i + 1])
        else:
            nk, nw = None, (params['W_out'], params['b_out'].reshape(1, -1))
        hV, T = _layer_call(True, nk, 4 * HID, hV, hE, G, eidx,
                            s_col if nk == 'dec' else None,
                            params['W_s'] if nk == 'dec' else None,
                            hVe if nk == 'dec' else None, lp, nw)
    return T


# speedup vs baseline: 3.0325x; 1.1368x over previous
import functools
import numpy as np
import jax
import jax.numpy as jnp
from jax import lax
from jax.experimental import pallas as pl
from jax.experimental.pallas import tpu as pltpu
from jax.experimental.pallas import tpu_sc as plsc

B, N, K = 4, 512, 30
HID = 128
NH, DH = 4, 32
NPE, NRBF = 16, 16
VOCAB = 20
SLAB = 128
NSLAB = N // SLAB

_PREC = lax.Precision.HIGHEST


def _dot(a, b):
    return lax.dot_general(a, b, (((1,), (0,)), ((), ())), precision=_PREC,
                           preferred_element_type=jnp.float32)


def _ln_rows(x, g, b):
    n = x.shape[-1]
    mu = jnp.mean(x, -1, keepdims=True)
    d = x - mu
    var = jnp.sum(d * d, -1, keepdims=True) / (n - 1)
    sigma = jnp.sqrt(var + 1e-6)
    return g * d / (sigma + 1e-6) + b


def _normalize3(v, eps=1e-12):
    n = jnp.sqrt(jnp.sum(v * v, -1, keepdims=True))
    return v / jnp.maximum(n, eps)


def _cross(a, b):
    ax, ay, az = a[:, 0:1], a[:, 1:2], a[:, 2:3]
    bx, by, bz = b[:, 0:1], b[:, 1:2], b[:, 2:3]
    return jnp.concatenate([ay * bz - az * by, az * bx - ax * bz,
                            ax * by - ay * bx], 1)


def _dot3(a, b):
    return jnp.sum(a * b, -1, keepdims=True)


def _shift_up(z):
    return jnp.concatenate([z[1:], jnp.zeros((1, z.shape[1]), z.dtype)], 0)


def _shift_down(z, fill=0.0):
    return jnp.concatenate([jnp.full((1, z.shape[1]), fill, z.dtype), z[:-1]], 0)


def _quaternion_cols(R):
    Rxx, Ryy, Rzz = R[0], R[4], R[8]
    m1 = 0.5 * jnp.sqrt(jnp.abs(1.0 + Rxx - Ryy - Rzz) + 1e-10)
    m2 = 0.5 * jnp.sqrt(jnp.abs(1.0 - Rxx + Ryy - Rzz) + 1e-10)
    m3 = 0.5 * jnp.sqrt(jnp.abs(1.0 - Rxx - Ryy + Rzz) + 1e-10)
    s1 = jnp.sign(R[7] - R[5])
    s2 = jnp.sign(R[2] - R[6])
    s3 = jnp.sign(R[3] - R[1])
    w = jnp.sqrt(jax.nn.relu(1.0 + Rxx + Ryy + Rzz) + 1e-10) / 2.0
    q = jnp.concatenate([s1 * m1, s2 * m2, s3 * m3, w], 1)
    return _normalize3(q)


def _feature_body(x_ref, xcat_ref, node_w, node_b, nn_g, nn_b, edge_w, edge_b,
                  ne_g, ne_b, w_v, b_v, w_e, b_e, wkv0,
                  he_ref, hv_ref, eidx_ref, idxg_ref, idxd_ref, t_ref):
    x = x_ref[0]
    xcat = xcat_ref[0]
    xca = x[:, 3:6]

    D2 = jnp.zeros((N, N), jnp.float32)
    for c in range(3):
        dc = xca[:, c:c + 1] - xcat[c:c + 1, :]
        D2 = D2 + dc * dc
    D = jnp.sqrt(D2 + 1e-6)

    iota_l = lax.broadcasted_iota(jnp.int32, (N, N), 1)
    iota_k = lax.broadcasted_iota(jnp.int32, (N, K), 1)
    Dw = D
    Dn = jnp.zeros((N, K), jnp.float32)
    Ei = jnp.zeros((N, K), jnp.int32)
    for kk in range(K):
        m = jnp.min(Dw, axis=1, keepdims=True)
        sel = jnp.min(jnp.where(Dw == m, iota_l, N), axis=1, keepdims=True)
        Dn = jnp.where(iota_k == kk, m, Dn)
        Ei = jnp.where(iota_k == kk, sel, Ei)
        Dw = jnp.where(iota_l == sel, jnp.float32(np.inf), Dw)
    eidx_ref[0] = Ei
    idxg_ref[0] = Ei + pl.program_id(0) * N
    rows_k = lax.broadcasted_iota(jnp.int32, (N, K), 0)
    idxd_ref[0] = (Ei + pl.program_id(0) * N
                   + jnp.where(Ei >= rows_k, B * N, 0))

    nA, cA, cC = x[:, 0:3], x[:, 3:6], x[:, 6:9]
    vA = cA - nA
    vB = cC - cA
    nA1 = _shift_up(nA)
    vC = nA1 - cC
    uA = _normalize3(vA)
    uB = _normalize3(vB)
    uC = _normalize3(vC)
    uA1 = _shift_up(uA)
    uB1 = _shift_up(uB)

    def dihed_cs(u2, u1, u0):
        n2 = _normalize3(_cross(u2, u1))
        n1 = _normalize3(_cross(u1, u0))
        cosD = jnp.clip(_dot3(n2, n1), -1.0 + 1e-7, 1.0 - 1e-7)
        sinD = jnp.sign(_dot3(u2, n1)) * jnp.sqrt(1.0 - cosD * cosD)
        return cosD, sinD

    rows = lax.broadcasted_iota(jnp.int32, (N, 1), 0)
    c1, s1 = dihed_cs(uA, uB, uC)
    c2, s2 = dihed_cs(uB, uC, uA1)
    c3, s3 = dihed_cs(uC, uA1, uB1)
    ok = rows <= N - 2
    c1 = jnp.where(ok, c1, 1.0)
    s1 = jnp.where(ok, s1, 0.0)
    c2 = jnp.where(ok, c2, 1.0)
    s2 = jnp.where(ok, s2, 0.0)
    c0 = _shift_down(c3, 1.0)
    s0 = _shift_down(s3, 0.0)
    Vf = jnp.concatenate([c0, c1, c2, s0, s1, s2], 1)

    xca1 = _shift_up(xca)
    Uc = _normalize3(xca1 - xca)
    Uc1 = _shift_up(Uc)
    o1 = _normalize3(Uc - Uc1)
    n2v = _normalize3(_cross(Uc, Uc1))
    o3 = _cross(o1, n2v)
    om9_raw = jnp.concatenate([o1, n2v, o3], 1)
    om9_sh = _shift_down(om9_raw, 0.0)
    okr = jnp.logical_and(rows >= 1, rows <= N - 3)
    Om9 = jnp.where(okr, om9_sh, 0.0)

    TBL = jnp.concatenate([xca, Om9], 1)

    iota8 = lax.broadcasted_iota(jnp.int32, (1, NPE // 2), 1).astype(jnp.float32)
    freq = jnp.exp(iota8 * jnp.float32(-2.0 * np.log(10000.0) / NPE))
    iota16 = lax.broadcasted_iota(jnp.int32, (1, NRBF), 1).astype(jnp.float32)
    mu = iota16 * jnp.float32(20.0 / (NRBF - 1))
    inv_sig = jnp.float32(NRBF / 20.0)
    n_f = rows.astype(jnp.float32)

    for k in range(K):
        sel = Ei[:, k:k + 1]
        dk = sel.astype(jnp.float32) - n_f
        ang = dk * freq
        epos = jnp.concatenate([jnp.cos(ang), jnp.sin(ang)], 1)
        dd = Dn[:, k:k + 1]
        t = (dd - mu) * inv_sig
        rbf = jnp.exp(-t * t)

        oh = (iota_l == sel).astype(jnp.float32)
        Gk = _dot(oh, TBL)
        xn = Gk[:, 0:3]
        on9 = Gk[:, 3:12]
        dXn = xn - xca
        du_cols = []
        for i in range(3):
            acc = (Om9[:, 3 * i:3 * i + 1] * dXn[:, 0:1]
                   + Om9[:, 3 * i + 1:3 * i + 2] * dXn[:, 1:2]
                   + Om9[:, 3 * i + 2:3 * i + 3] * dXn[:, 2:3])
            du_cols.append(acc)
        du = _normalize3(jnp.concatenate(du_cols, 1))
        Rcols = []
        for i in range(3):
            for l in range(3):
                r = (Om9[:, 0 + i:1 + i] * on9[:, 0 + l:1 + l]
                     + Om9[:, 3 + i:4 + i] * on9[:, 3 + l:4 + l]
                     + Om9[:, 6 + i:7 + i] * on9[:, 6 + l:7 + l])
                Rcols.append(r)
        q = _quaternion_cols(Rcols)
        of_k = jnp.concatenate([du, q], 1)

        e_k = jnp.concatenate([epos, rbf, of_k], 1)
        he = _ln_rows(_dot(e_k, edge_w[...]) + edge_b[...], ne_g[...], ne_b[...])
        he_ref[0, :, HID * k:HID * (k + 1)] = _dot(he, w_e[...]) + b_e[...]

    v = _ln_rows(_dot(Vf, node_w[...]) + node_b[...], nn_g[...], nn_b[...])
    hv = _dot(v, w_v[...]) + b_v[...]
    hv_ref[0] = hv
    t_ref[0] = _dot(hv, wkv0[...])


def _layer_body(is_dec, next_kind, C, *refs):
    it = iter(refs)
    hv_ref = next(it)
    he_ref = next(it)
    g_ref = next(it)
    if next_kind == 'dec':
        s_ref = next(it)
        w_s = next(it)
        hve_ref = next(it) if is_dec else None
    wq = next(it)
    wkv_e = next(it)
    wo = next(it)
    n0g = next(it)
    n0b = next(it)
    wi = next(it)
    bi = next(it)
    wo2 = next(it)
    bo = next(it)
    n1g = next(it)
    n1b = next(it)
    if next_kind == 'enc':
        wnext = next(it)
    elif next_kind == 'dec':
        wnA = next(it)
        wnB = next(it)
    else:
        wout = next(it)
        bout = next(it)
    hv_out = next(it)
    t_out = next(it)

    hv = hv_ref[0]
    he = he_ref[0]
    g = g_ref[0]

    q = _dot(hv, wq[...])
    scale = jnp.float32(1.0 / np.sqrt(DH))
    iota_k = lax.broadcasted_iota(jnp.int32, (SLAB, K), 1)
    lgs = [jnp.zeros((SLAB, K), jnp.float32) for _ in range(NH)]
    vbuf = []
    for k in range(K):
        ek = he[:, HID * k:HID * (k + 1)]
        kv = _dot(ek, wkv_e[...])
        kcol = kv[:, :HID]
        vcol = kv[:, HID:]
        base = C * k
        kk = kcol + g[:, base:base + HID]
        vv = vcol + g[:, base + HID:base + 2 * HID]
        vbuf.append(vv)
        for h in range(NH):
            sl = slice(DH * h, DH * (h + 1))
            lh = jnp.sum(q[:, sl] * kk[:, sl], axis=1, keepdims=True)
            lgs[h] = jnp.where(iota_k == k, lh * scale, lgs[h])

    accs = []
    for h in range(NH):
        lg = lgs[h]
        m = jnp.max(lg, 1, keepdims=True)
        e = jnp.exp(lg - m)
        a = e / jnp.sum(e, 1, keepdims=True)
        acc = jnp.zeros((SLAB, DH), jnp.float32)
        for k in range(K):
            acc = acc + a[:, k:k + 1] * vbuf[k][:, DH * h:DH * (h + 1)]
        accs.append(acc)
    upd = _dot(jnp.concatenate(accs, 1), wo[...])

    h1 = _ln_rows(hv + upd, n0g[...], n0b[...])
    ffn = _dot(jax.nn.relu(_dot(h1, wi[...]) + bi[...]), wo2[...]) + bo[...]
    h2 = _ln_rows(h1 + ffn, n1g[...], n1b[...])
    hv_out[0] = h2

    if next_kind == 'enc':
        t_out[0] = _dot(h2, wnext[...])
    elif next_kind == 'dec':
        s_col = s_ref[0]
        iota20 = lax.broadcasted_iota(jnp.int32, (1, VOCAB), 1)
        oh_s = (s_col == iota20).astype(jnp.float32)
        hs = _dot(oh_s, w_s[...])
        hve = h2 if not is_dec else hve_ref[0]
        t_out[0] = jnp.concatenate(
            [_dot(hs, wnA[...]) + _dot(h2, wnB[...]), _dot(hve, wnB[...])], 1)
    else:
        lg = _dot(h2, wout[...]) + bout[...]
        m = jnp.max(lg, 1, keepdims=True)
        t_out[0] = lg - m - jnp.log(jnp.sum(jnp.exp(lg - m), 1, keepdims=True))


def _full_spec(shape):
    nd = len(shape)
    return pl.BlockSpec(shape, lambda b, s, _n=nd: (0,) * _n)


def _slab_spec(f):
    return pl.BlockSpec((1, SLAB, f), lambda b, s: (b, s, 0))


def _feature_call(Xr, XcaT, fp, w_v, b_v, w_e, b_e, wkv0):
    in_specs = [pl.BlockSpec((1, N, 12), lambda b: (b, 0, 0)),
                pl.BlockSpec((1, 3, N), lambda b: (b, 0, 0))]
    weights = [fp['node_W'], fp['node_b'].reshape(1, -1), fp['nn_g'].reshape(1, -1),
               fp['nn_b'].reshape(1, -1), fp['edge_W'], fp['edge_b'].reshape(1, -1),
               fp['ne_g'].reshape(1, -1), fp['ne_b'].reshape(1, -1),
               w_v, b_v.reshape(1, -1), w_e, b_e.reshape(1, -1), wkv0]
    for w in weights:
        in_specs.append(pl.BlockSpec(w.shape, lambda b, _n=len(w.shape): (0,) * _n))
    out_shape = [jax.ShapeDtypeStruct((B, N, K * HID), jnp.float32),
                 jax.ShapeDtypeStruct((B, N, HID), jnp.float32),
                 jax.ShapeDtypeStruct((B, N, K), jnp.int32),
                 jax.ShapeDtypeStruct((B, N, K), jnp.int32),
                 jax.ShapeDtypeStruct((B, N, K), jnp.int32),
                 jax.ShapeDtypeStruct((B, N, 2 * HID), jnp.float32)]
    out_specs = [pl.BlockSpec((1, N, K * HID), lambda b: (b, 0, 0)),
                 pl.BlockSpec((1, N, HID), lambda b: (b, 0, 0)),
                 pl.BlockSpec((1, N, K), lambda b: (b, 0, 0)),
                 pl.BlockSpec((1, N, K), lambda b: (b, 0, 0)),
                 pl.BlockSpec((1, N, K), lambda b: (b, 0, 0)),
                 pl.BlockSpec((1, N, 2 * HID), lambda b: (b, 0, 0))]
    return pl.pallas_call(
        _feature_body, grid=(B,), in_specs=in_specs, out_specs=out_specs,
        out_shape=out_shape)(Xr, XcaT, *weights)


def _layer_call(is_dec, next_kind, C, hv, he, g, s_col, w_s, hve, lp,
                next_w):
    args = [hv, he, g]
    in_specs = [_slab_spec(HID), _slab_spec(K * HID), _slab_spec(K * C)]
    if next_kind == 'dec':
        args.append(s_col)
        in_specs.append(_slab_spec(1))
        args.append(w_s)
        in_specs.append(_full_spec(w_s.shape))
        if is_dec:
            args.append(hve)
            in_specs.append(_slab_spec(HID))
    weights = [lp['WQ'], jnp.concatenate([lp['WK'][:HID], lp['WV'][:HID]], 1),
               lp['WO'], lp['n0_g'].reshape(1, -1), lp['n0_b'].reshape(1, -1),
               lp['Wi'], lp['bi'].reshape(1, -1), lp['Wo'],
               lp['bo'].reshape(1, -1), lp['n1_g'].reshape(1, -1),
               lp['n1_b'].reshape(1, -1)]
    weights += [w for w in next_w]
    for w in weights:
        args.append(w)
        in_specs.append(_full_spec(w.shape))

    out_shape = [jax.ShapeDtypeStruct((B, N, HID), jnp.float32)]
    out_specs = [_slab_spec(HID)]
    if next_kind == 'enc':
        out_shape.append(jax.ShapeDtypeStruct((B, N, 2 * HID), jnp.float32))
        out_specs.append(_slab_spec(2 * HID))
    elif next_kind == 'dec':
        out_shape.append(jax.ShapeDtypeStruct((B, N, 4 * HID), jnp.float32))
        out_specs.append(_slab_spec(4 * HID))
    else:
        out_shape.append(jax.ShapeDtypeStruct((B, N, VOCAB), jnp.float32))
        out_specs.append(_slab_spec(VOCAB))

    body = functools.partial(_layer_body, is_dec, next_kind, C)
    return pl.pallas_call(
        body, grid=(B, NSLAB), in_specs=in_specs, out_specs=out_specs,
        out_shape=out_shape)(*args)


def _sc_gather(table, idx, C):
    M = idx.shape[0]
    NW = 32
    per_w = M // NW
    chunk = 384 if C <= 256 else 192
    n_it = per_w // chunk
    mesh = plsc.VectorSubcoreMesh(core_axis_name="c", subcore_axis_name="s")

    def body(table_ref, idx_ref, out_ref, idx_v, rows_v, sem):
        wid = lax.axis_index("s") * 2 + lax.axis_index("c")
        base = wid * per_w

        def it(i, carry):
            off = base + i * chunk
            pltpu.sync_copy(idx_ref.at[pl.ds(off, chunk)], idx_v)
            pltpu.async_copy(table_ref.at[idx_v], rows_v, sem).wait()
            pltpu.sync_copy(rows_v, out_ref.at[pl.ds(off, chunk)])
            return carry

        lax.fori_loop(0, n_it, it, 0)

    f = pl.kernel(body,
                  out_type=jax.ShapeDtypeStruct((M, C), jnp.float32),
                  mesh=mesh,
                  scratch_types=[pltpu.VMEM((chunk,), jnp.int32),
                                 pltpu.VMEM((chunk, C), jnp.float32),
                                 pltpu.SemaphoreType.DMA])
    return f(table, idx)


_gather = _sc_gather


def kernel(X, S, L, mask, params):
    fp = params['feat']
    enc = params['enc']
    dec = params['dec']
    Xr = X.reshape(B, N, 12)
    XcaT = jnp.transpose(X[:, :, 1, :], (0, 2, 1))
    s_col = S.reshape(B, N, 1).astype(jnp.int32)

    def enc_tbl_w(lp):
        return jnp.concatenate([lp['WK'][HID:], lp['WV'][HID:]], 1)

    def dec_tbl_w(lp):
        wnA = jnp.concatenate([lp['WK'][HID:2 * HID], lp['WV'][HID:2 * HID]], 1)
        wnB = jnp.concatenate([lp['WK'][2 * HID:], lp['WV'][2 * HID:]], 1)
        return wnA, wnB

    hE, hV, eidx, idxg, idxd, T = _feature_call(
        Xr, XcaT, fp, params['W_v'], params['b_v'], params['W_e'],
        params['b_e'], enc_tbl_w(enc[0]))
    idx_flat = idxg.reshape(B * N * K)
    idxd_flat = idxd.reshape(B * N * K)

    def dec_table(T4):
        return jnp.concatenate([T4[:, :, :2 * HID].reshape(B * N, 2 * HID),
                                T4[:, :, 2 * HID:].reshape(B * N, 2 * HID)], 0)

    for i in range(3):
        lp = enc[i]
        G = _gather(T.reshape(B * N, 2 * HID), idx_flat, 2 * HID)
        G = G.reshape(B, N, K * 2 * HID)
        if i < 2:
            nk, nw = 'enc', (enc_tbl_w(enc[i + 1]),)
        else:
            nk, nw = 'dec', dec_tbl_w(dec[0])
        hV, T = _layer_call(False, nk, 2 * HID, hV, hE, G,
                            s_col if nk == 'dec' else None,
                            params['W_s'] if nk == 'dec' else None,
                            None, lp, nw)

    hVe = hV
    for i in range(3):
        lp = dec[i]
        G = _gather(dec_table(T), idxd_flat, 2 * HID)
        G = G.reshape(B, N, K * 2 * HID)
        if i < 2:
            nk, nw = 'dec', dec_tbl_w(dec[i + 1])
        else:
            nk, nw = None, (params['W_out'], params['b_out'].reshape(1, -1))
        hV, T = _layer_call(True, nk, 2 * HID, hV, hE, G,
                            s_col if nk == 'dec' else None,
                            params['W_s'] if nk == 'dec' else None,
                            hVe if nk == 'dec' else None, lp, nw)
    return T


# speedup vs baseline: 3.2899x; 1.0849x over previous
import functools
import numpy as np
import jax
import jax.numpy as jnp
from jax import lax
from jax.experimental import pallas as pl
from jax.experimental.pallas import tpu as pltpu
from jax.experimental.pallas import tpu_sc as plsc

B, N, K = 4, 512, 30
HID = 128
NH, DH = 4, 32
NPE, NRBF = 16, 16
VOCAB = 20
SLAB = 256
NSLAB = N // SLAB

_PREC = lax.Precision.HIGHEST


def _dot(a, b):
    return lax.dot_general(a, b, (((1,), (0,)), ((), ())), precision=_PREC,
                           preferred_element_type=jnp.float32)


def _ln_rows(x, g, b):
    n = x.shape[-1]
    mu = jnp.mean(x, -1, keepdims=True)
    d = x - mu
    var = jnp.sum(d * d, -1, keepdims=True) / (n - 1)
    sigma = jnp.sqrt(var + 1e-6)
    return g * d / (sigma + 1e-6) + b


def _normalize3(v, eps=1e-12):
    n = jnp.sqrt(jnp.sum(v * v, -1, keepdims=True))
    return v / jnp.maximum(n, eps)


def _cross(a, b):
    ax, ay, az = a[:, 0:1], a[:, 1:2], a[:, 2:3]
    bx, by, bz = b[:, 0:1], b[:, 1:2], b[:, 2:3]
    return jnp.concatenate([ay * bz - az * by, az * bx - ax * bz,
                            ax * by - ay * bx], 1)


def _dot3(a, b):
    return jnp.sum(a * b, -1, keepdims=True)


def _shift_up(z):
    return jnp.concatenate([z[1:], jnp.zeros((1, z.shape[1]), z.dtype)], 0)


def _shift_down(z, fill=0.0):
    return jnp.concatenate([jnp.full((1, z.shape[1]), fill, z.dtype), z[:-1]], 0)


def _quaternion_cols(R):
    Rxx, Ryy, Rzz = R[0], R[4], R[8]
    m1 = 0.5 * jnp.sqrt(jnp.abs(1.0 + Rxx - Ryy - Rzz) + 1e-10)
    m2 = 0.5 * jnp.sqrt(jnp.abs(1.0 - Rxx + Ryy - Rzz) + 1e-10)
    m3 = 0.5 * jnp.sqrt(jnp.abs(1.0 - Rxx - Ryy + Rzz) + 1e-10)
    s1 = jnp.sign(R[7] - R[5])
    s2 = jnp.sign(R[2] - R[6])
    s3 = jnp.sign(R[3] - R[1])
    w = jnp.sqrt(jax.nn.relu(1.0 + Rxx + Ryy + Rzz) + 1e-10) / 2.0
    q = jnp.concatenate([s1 * m1, s2 * m2, s3 * m3, w], 1)
    return _normalize3(q)


def _feature_body(x_ref, xcat_ref, node_w, node_b, nn_g, nn_b, edge_w, edge_b,
                  ne_g, ne_b, w_v, b_v, w_e, b_e, wkv0,
                  he_ref, hv_ref, eidx_ref, idxg_ref, idxd_ref, t_ref):
    x = x_ref[0]
    xcat = xcat_ref[0]
    xca = x[:, 3:6]

    D2 = jnp.zeros((N, N), jnp.float32)
    for c in range(3):
        dc = xca[:, c:c + 1] - xcat[c:c + 1, :]
        D2 = D2 + dc * dc
    D = jnp.sqrt(D2 + 1e-6)

    iota_l = lax.broadcasted_iota(jnp.int32, (N, N), 1)
    iota_k = lax.broadcasted_iota(jnp.int32, (N, K), 1)
    Dw = D
    Dn = jnp.zeros((N, K), jnp.float32)
    Ei = jnp.zeros((N, K), jnp.int32)
    for kk in range(K):
        m = jnp.min(Dw, axis=1, keepdims=True)
        sel = jnp.min(jnp.where(Dw == m, iota_l, N), axis=1, keepdims=True)
        Dn = jnp.where(iota_k == kk, m, Dn)
        Ei = jnp.where(iota_k == kk, sel, Ei)
        Dw = jnp.where(iota_l == sel, jnp.float32(np.inf), Dw)
    eidx_ref[0] = Ei
    idxg_ref[0] = Ei + pl.program_id(0) * N
    rows_k = lax.broadcasted_iota(jnp.int32, (N, K), 0)
    idxd_ref[0] = (Ei + pl.program_id(0) * N
                   + jnp.where(Ei >= rows_k, B * N, 0))

    nA, cA, cC = x[:, 0:3], x[:, 3:6], x[:, 6:9]
    vA = cA - nA
    vB = cC - cA
    nA1 = _shift_up(nA)
    vC = nA1 - cC
    uA = _normalize3(vA)
    uB = _normalize3(vB)
    uC = _normalize3(vC)
    uA1 = _shift_up(uA)
    uB1 = _shift_up(uB)

    def dihed_cs(u2, u1, u0):
        n2 = _normalize3(_cross(u2, u1))
        n1 = _normalize3(_cross(u1, u0))
        cosD = jnp.clip(_dot3(n2, n1), -1.0 + 1e-7, 1.0 - 1e-7)
        sinD = jnp.sign(_dot3(u2, n1)) * jnp.sqrt(1.0 - cosD * cosD)
        return cosD, sinD

    rows = lax.broadcasted_iota(jnp.int32, (N, 1), 0)
    c1, s1 = dihed_cs(uA, uB, uC)
    c2, s2 = dihed_cs(uB, uC, uA1)
    c3, s3 = dihed_cs(uC, uA1, uB1)
    ok = rows <= N - 2
    c1 = jnp.where(ok, c1, 1.0)
    s1 = jnp.where(ok, s1, 0.0)
    c2 = jnp.where(ok, c2, 1.0)
    s2 = jnp.where(ok, s2, 0.0)
    c0 = _shift_down(c3, 1.0)
    s0 = _shift_down(s3, 0.0)
    Vf = jnp.concatenate([c0, c1, c2, s0, s1, s2], 1)

    xca1 = _shift_up(xca)
    Uc = _normalize3(xca1 - xca)
    Uc1 = _shift_up(Uc)
    o1 = _normalize3(Uc - Uc1)
    n2v = _normalize3(_cross(Uc, Uc1))
    o3 = _cross(o1, n2v)
    om9_raw = jnp.concatenate([o1, n2v, o3], 1)
    om9_sh = _shift_down(om9_raw, 0.0)
    okr = jnp.logical_and(rows >= 1, rows <= N - 3)
    Om9 = jnp.where(okr, om9_sh, 0.0)

    TBL = jnp.concatenate([xca, Om9], 1)

    iota8 = lax.broadcasted_iota(jnp.int32, (1, NPE // 2), 1).astype(jnp.float32)
    freq = jnp.exp(iota8 * jnp.float32(-2.0 * np.log(10000.0) / NPE))
    iota16 = lax.broadcasted_iota(jnp.int32, (1, NRBF), 1).astype(jnp.float32)
    mu = iota16 * jnp.float32(20.0 / (NRBF - 1))
    inv_sig = jnp.float32(NRBF / 20.0)
    n_f = rows.astype(jnp.float32)

    for k in range(K):
        sel = Ei[:, k:k + 1]
        dk = sel.astype(jnp.float32) - n_f
        ang = dk * freq
        epos = jnp.concatenate([jnp.cos(ang), jnp.sin(ang)], 1)
        dd = Dn[:, k:k + 1]
        t = (dd - mu) * inv_sig
        rbf = jnp.exp(-t * t)

        oh = (iota_l == sel).astype(jnp.float32)
        Gk = _dot(oh, TBL)
        xn = Gk[:, 0:3]
        on9 = Gk[:, 3:12]
        dXn = xn - xca
        du_cols = []
        for i in range(3):
            acc = (Om9[:, 3 * i:3 * i + 1] * dXn[:, 0:1]
                   + Om9[:, 3 * i + 1:3 * i + 2] * dXn[:, 1:2]
                   + Om9[:, 3 * i + 2:3 * i + 3] * dXn[:, 2:3])
            du_cols.append(acc)
        du = _normalize3(jnp.concatenate(du_cols, 1))
        Rcols = []
        for i in range(3):
            for l in range(3):
                r = (Om9[:, 0 + i:1 + i] * on9[:, 0 + l:1 + l]
                     + Om9[:, 3 + i:4 + i] * on9[:, 3 + l:4 + l]
                     + Om9[:, 6 + i:7 + i] * on9[:, 6 + l:7 + l])
                Rcols.append(r)
        q = _quaternion_cols(Rcols)
        of_k = jnp.concatenate([du, q], 1)

        e_k = jnp.concatenate([epos, rbf, of_k], 1)
        he = _ln_rows(_dot(e_k, edge_w[...]) + edge_b[...], ne_g[...], ne_b[...])
        he_ref[0, :, HID * k:HID * (k + 1)] = _dot(he, w_e[...]) + b_e[...]

    v = _ln_rows(_dot(Vf, node_w[...]) + node_b[...], nn_g[...], nn_b[...])
    hv = _dot(v, w_v[...]) + b_v[...]
    hv_ref[0] = hv
    t_ref[0] = _dot(hv, wkv0[...])


def _layer_body(is_dec, next_kind, C, *refs):
    it = iter(refs)
    hv_ref = next(it)
    he_ref = next(it)
    g_ref = next(it)
    if next_kind == 'dec':
        s_ref = next(it)
        w_s = next(it)
        hve_ref = next(it) if is_dec else None
    wq = next(it)
    wkv_e = next(it)
    wo = next(it)
    n0g = next(it)
    n0b = next(it)
    wi = next(it)
    bi = next(it)
    wo2 = next(it)
    bo = next(it)
    n1g = next(it)
    n1b = next(it)
    if next_kind == 'enc':
        wnext = next(it)
    elif next_kind == 'dec':
        wnA = next(it)
        wnB = next(it)
    else:
        wout = next(it)
        bout = next(it)
    hv_out = next(it)
    t_out = next(it)

    hv = hv_ref[0]
    he = he_ref[0]
    g = g_ref[0]

    q = _dot(hv, wq[...])
    scale = jnp.float32(1.0 / np.sqrt(DH))
    iota_k = lax.broadcasted_iota(jnp.int32, (SLAB, K), 1)
    lgs = [jnp.zeros((SLAB, K), jnp.float32) for _ in range(NH)]
    vbuf = []
    for k in range(K):
        ek = he[:, HID * k:HID * (k + 1)]
        kv = _dot(ek, wkv_e[...])
        kcol = kv[:, :HID]
        vcol = kv[:, HID:]
        base = C * k
        kk = kcol + g[:, base:base + HID]
        vv = vcol + g[:, base + HID:base + 2 * HID]
        vbuf.append(vv)
        for h in range(NH):
            sl = slice(DH * h, DH * (h + 1))
            lh = jnp.sum(q[:, sl] * kk[:, sl], axis=1, keepdims=True)
            lgs[h] = jnp.where(iota_k == k, lh * scale, lgs[h])

    accs = []
    for h in range(NH):
        lg = lgs[h]
        m = jnp.max(lg, 1, keepdims=True)
        e = jnp.exp(lg - m)
        a = e / jnp.sum(e, 1, keepdims=True)
        acc = jnp.zeros((SLAB, DH), jnp.float32)
        for k in range(K):
            acc = acc + a[:, k:k + 1] * vbuf[k][:, DH * h:DH * (h + 1)]
        accs.append(acc)
    upd = _dot(jnp.concatenate(accs, 1), wo[...])

    h1 = _ln_rows(hv + upd, n0g[...], n0b[...])
    ffn = _dot(jax.nn.relu(_dot(h1, wi[...]) + bi[...]), wo2[...]) + bo[...]
    h2 = _ln_rows(h1 + ffn, n1g[...], n1b[...])
    hv_out[0] = h2

    if next_kind == 'enc':
        t_out[0] = _dot(h2, wnext[...])
    elif next_kind == 'dec':
        s_col = s_ref[0]
        iota20 = lax.broadcasted_iota(jnp.int32, (1, VOCAB), 1)
        oh_s = (s_col == iota20).astype(jnp.float32)
        hs = _dot(oh_s, w_s[...])
        hve = h2 if not is_dec else hve_ref[0]
        t_out[0] = jnp.concatenate(
            [_dot(hs, wnA[...]) + _dot(h2, wnB[...]), _dot(hve, wnB[...])], 1)
    else:
        lg = _dot(h2, wout[...]) + bout[...]
        m = jnp.max(lg, 1, keepdims=True)
        t_out[0] = lg - m - jnp.log(jnp.sum(jnp.exp(lg - m), 1, keepdims=True))


def _full_spec(shape):
    nd = len(shape)
    return pl.BlockSpec(shape, lambda b, s, _n=nd: (0,) * _n)


def _slab_spec(f):
    return pl.BlockSpec((1, SLAB, f), lambda b, s: (b, s, 0))


def _feature_call(Xr, XcaT, fp, w_v, b_v, w_e, b_e, wkv0):
    in_specs = [pl.BlockSpec((1, N, 12), lambda b: (b, 0, 0)),
                pl.BlockSpec((1, 3, N), lambda b: (b, 0, 0))]
    weights = [fp['node_W'], fp['node_b'].reshape(1, -1), fp['nn_g'].reshape(1, -1),
               fp['nn_b'].reshape(1, -1), fp['edge_W'], fp['edge_b'].reshape(1, -1),
               fp['ne_g'].reshape(1, -1), fp['ne_b'].reshape(1, -1),
               w_v, b_v.reshape(1, -1), w_e, b_e.reshape(1, -1), wkv0]
    for w in weights:
        in_specs.append(pl.BlockSpec(w.shape, lambda b, _n=len(w.shape): (0,) * _n))
    out_shape = [jax.ShapeDtypeStruct((B, N, K * HID), jnp.float32),
                 jax.ShapeDtypeStruct((B, N, HID), jnp.float32),
                 jax.ShapeDtypeStruct((B, N, K), jnp.int32),
                 jax.ShapeDtypeStruct((B, N, K), jnp.int32),
                 jax.ShapeDtypeStruct((B, N, K), jnp.int32),
                 jax.ShapeDtypeStruct((B, N, 2 * HID), jnp.float32)]
    out_specs = [pl.BlockSpec((1, N, K * HID), lambda b: (b, 0, 0)),
                 pl.BlockSpec((1, N, HID), lambda b: (b, 0, 0)),
                 pl.BlockSpec((1, N, K), lambda b: (b, 0, 0)),
                 pl.BlockSpec((1, N, K), lambda b: (b, 0, 0)),
                 pl.BlockSpec((1, N, K), lambda b: (b, 0, 0)),
                 pl.BlockSpec((1, N, 2 * HID), lambda b: (b, 0, 0))]
    return pl.pallas_call(
        _feature_body, grid=(B,), in_specs=in_specs, out_specs=out_specs,
        out_shape=out_shape)(Xr, XcaT, *weights)


def _layer_call(is_dec, next_kind, C, hv, he, g, s_col, w_s, hve, lp,
                next_w):
    args = [hv, he, g]
    in_specs = [_slab_spec(HID), _slab_spec(K * HID), _slab_spec(K * C)]
    if next_kind == 'dec':
        args.append(s_col)
        in_specs.append(_slab_spec(1))
        args.append(w_s)
        in_specs.append(_full_spec(w_s.shape))
        if is_dec:
            args.append(hve)
            in_specs.append(_slab_spec(HID))
    weights = [lp['WQ'], jnp.concatenate([lp['WK'][:HID], lp['WV'][:HID]], 1),
               lp['WO'], lp['n0_g'].reshape(1, -1), lp['n0_b'].reshape(1, -1),
               lp['Wi'], lp['bi'].reshape(1, -1), lp['Wo'],
               lp['bo'].reshape(1, -1), lp['n1_g'].reshape(1, -1),
               lp['n1_b'].reshape(1, -1)]
    weights += [w for w in next_w]
    for w in weights:
        args.append(w)
        in_specs.append(_full_spec(w.shape))

    out_shape = [jax.ShapeDtypeStruct((B, N, HID), jnp.float32)]
    out_specs = [_slab_spec(HID)]
    if next_kind == 'enc':
        out_shape.append(jax.ShapeDtypeStruct((B, N, 2 * HID), jnp.float32))
        out_specs.append(_slab_spec(2 * HID))
    elif next_kind == 'dec':
        out_shape.append(jax.ShapeDtypeStruct((B, N, 4 * HID), jnp.float32))
        out_specs.append(_slab_spec(4 * HID))
    else:
        out_shape.append(jax.ShapeDtypeStruct((B, N, VOCAB), jnp.float32))
        out_specs.append(_slab_spec(VOCAB))

    body = functools.partial(_layer_body, is_dec, next_kind, C)
    return pl.pallas_call(
        body, grid=(B, NSLAB), in_specs=in_specs, out_specs=out_specs,
        out_shape=out_shape)(*args)


def _sc_gather(table, idx, C):
    M = idx.shape[0]
    NW = 32
    per_w = M // NW
    chunk = 384 if C <= 256 else 192
    n_it = per_w // chunk
    mesh = plsc.VectorSubcoreMesh(core_axis_name="c", subcore_axis_name="s")

    def body(table_ref, idx_ref, out_ref, idx_v, rows_v, sem):
        wid = lax.axis_index("s") * 2 + lax.axis_index("c")
        base = wid * per_w

        def it(i, carry):
            off = base + i * chunk
            pltpu.sync_copy(idx_ref.at[pl.ds(off, chunk)], idx_v)
            pltpu.async_copy(table_ref.at[idx_v], rows_v, sem).wait()
            pltpu.sync_copy(rows_v, out_ref.at[pl.ds(off, chunk)])
            return carry

        lax.fori_loop(0, n_it, it, 0)

    f = pl.kernel(body,
                  out_type=jax.ShapeDtypeStruct((M, C), jnp.float32),
                  mesh=mesh,
                  scratch_types=[pltpu.VMEM((chunk,), jnp.int32),
                                 pltpu.VMEM((chunk, C), jnp.float32),
                                 pltpu.SemaphoreType.DMA])
    return f(table, idx)


_gather = _sc_gather


def kernel(X, S, L, mask, params):
    fp = params['feat']
    enc = params['enc']
    dec = params['dec']
    Xr = X.reshape(B, N, 12)
    XcaT = jnp.transpose(X[:, :, 1, :], (0, 2, 1))
    s_col = S.reshape(B, N, 1).astype(jnp.int32)

    def enc_tbl_w(lp):
        return jnp.concatenate([lp['WK'][HID:], lp['WV'][HID:]], 1)

    def dec_tbl_w(lp):
        wnA = jnp.concatenate([lp['WK'][HID:2 * HID], lp['WV'][HID:2 * HID]], 1)
        wnB = jnp.concatenate([lp['WK'][2 * HID:], lp['WV'][2 * HID:]], 1)
        return wnA, wnB

    hE, hV, eidx, idxg, idxd, T = _feature_call(
        Xr, XcaT, fp, params['W_v'], params['b_v'], params['W_e'],
        params['b_e'], enc_tbl_w(enc[0]))
    idx_flat = idxg.reshape(B * N * K)
    idxd_flat = idxd.reshape(B * N * K)

    def dec_table(T4):
        return jnp.concatenate([T4[:, :, :2 * HID].reshape(B * N, 2 * HID),
                                T4[:, :, 2 * HID:].reshape(B * N, 2 * HID)], 0)

    for i in range(3):
        lp = enc[i]
        G = _gather(T.reshape(B * N, 2 * HID), idx_flat, 2 * HID)
        G = G.reshape(B, N, K * 2 * HID)
        if i < 2:
            nk, nw = 'enc', (enc_tbl_w(enc[i + 1]),)
        else:
            nk, nw = 'dec', dec_tbl_w(dec[0])
        hV, T = _layer_call(False, nk, 2 * HID, hV, hE, G,
                            s_col if nk == 'dec' else None,
                            params['W_s'] if nk == 'dec' else None,
                            None, lp, nw)

    hVe = hV
    for i in range(3):
        lp = dec[i]
        G = _gather(dec_table(T), idxd_flat, 2 * HID)
        G = G.reshape(B, N, K * 2 * HID)
        if i < 2:
            nk, nw = 'dec', dec_tbl_w(dec[i + 1])
        else:
            nk, nw = None, (params['W_out'], params['b_out'].reshape(1, -1))
        hV, T = _layer_call(True, nk, 2 * HID, hV, hE, G,
                            s_col if nk == 'dec' else None,
                            params['W_s'] if nk == 'dec' else None,
                            hVe if nk == 'dec' else None, lp, nw)
    return T


# speedup vs baseline: 3.3574x; 1.0205x over previous
import functools
import numpy as np
import jax
import jax.numpy as jnp
from jax import lax
from jax.experimental import pallas as pl
from jax.experimental.pallas import tpu as pltpu
from jax.experimental.pallas import tpu_sc as plsc

B, N, K = 4, 512, 30
HID = 128
NH, DH = 4, 32
NPE, NRBF = 16, 16
VOCAB = 20
SLAB = 256
NSLAB = N // SLAB

_PREC = lax.Precision.HIGHEST


def _attn_mats():
    j = np.arange(K * HID)
    r = j % HID
    mcol = (r // DH) * K + (j // HID)
    m_red = np.zeros((K * HID, NH * K), np.float32)
    m_red[j, mcol] = 1.0
    m_sum = np.zeros((K * HID, HID), np.float32)
    m_sum[j, r] = 1.0
    return (jnp.asarray(m_red), jnp.asarray(np.ascontiguousarray(m_red.T)),
            jnp.asarray(m_sum))


_M_RED, _M_EXP, _M_SUM = _attn_mats()


def _dot(a, b):
    return lax.dot_general(a, b, (((1,), (0,)), ((), ())), precision=_PREC,
                           preferred_element_type=jnp.float32)


def _ln_rows(x, g, b):
    n = x.shape[-1]
    mu = jnp.mean(x, -1, keepdims=True)
    d = x - mu
    var = jnp.sum(d * d, -1, keepdims=True) / (n - 1)
    sigma = jnp.sqrt(var + 1e-6)
    return g * d / (sigma + 1e-6) + b


def _normalize3(v, eps=1e-12):
    n = jnp.sqrt(jnp.sum(v * v, -1, keepdims=True))
    return v / jnp.maximum(n, eps)


def _cross(a, b):
    ax, ay, az = a[:, 0:1], a[:, 1:2], a[:, 2:3]
    bx, by, bz = b[:, 0:1], b[:, 1:2], b[:, 2:3]
    return jnp.concatenate([ay * bz - az * by, az * bx - ax * bz,
                            ax * by - ay * bx], 1)


def _dot3(a, b):
    return jnp.sum(a * b, -1, keepdims=True)


def _shift_up(z):
    return jnp.concatenate([z[1:], jnp.zeros((1, z.shape[1]), z.dtype)], 0)


def _shift_down(z, fill=0.0):
    return jnp.concatenate([jnp.full((1, z.shape[1]), fill, z.dtype), z[:-1]], 0)


def _quaternion_cols(R):
    Rxx, Ryy, Rzz = R[0], R[4], R[8]
    m1 = 0.5 * jnp.sqrt(jnp.abs(1.0 + Rxx - Ryy - Rzz) + 1e-10)
    m2 = 0.5 * jnp.sqrt(jnp.abs(1.0 - Rxx + Ryy - Rzz) + 1e-10)
    m3 = 0.5 * jnp.sqrt(jnp.abs(1.0 - Rxx - Ryy + Rzz) + 1e-10)
    s1 = jnp.sign(R[7] - R[5])
    s2 = jnp.sign(R[2] - R[6])
    s3 = jnp.sign(R[3] - R[1])
    w = jnp.sqrt(jax.nn.relu(1.0 + Rxx + Ryy + Rzz) + 1e-10) / 2.0
    q = jnp.concatenate([s1 * m1, s2 * m2, s3 * m3, w], 1)
    return _normalize3(q)


def _feature_body(x_ref, xcat_ref, node_w, node_b, nn_g, nn_b, edge_w, edge_b,
                  ne_g, ne_b, w_v, b_v, w_e, b_e, wkv0,
                  he_ref, hv_ref, eidx_ref, idxg_ref, idxd_ref, t_ref):
    x = x_ref[0]
    xcat = xcat_ref[0]
    xca = x[:, 3:6]

    D2 = jnp.zeros((N, N), jnp.float32)
    for c in range(3):
        dc = xca[:, c:c + 1] - xcat[c:c + 1, :]
        D2 = D2 + dc * dc
    D = jnp.sqrt(D2 + 1e-6)

    iota_l = lax.broadcasted_iota(jnp.int32, (N, N), 1)
    iota_k = lax.broadcasted_iota(jnp.int32, (N, K), 1)
    Dw = D
    Dn = jnp.zeros((N, K), jnp.float32)
    Ei = jnp.zeros((N, K), jnp.int32)
    for kk in range(K):
        m = jnp.min(Dw, axis=1, keepdims=True)
        sel = jnp.min(jnp.where(Dw == m, iota_l, N), axis=1, keepdims=True)
        Dn = jnp.where(iota_k == kk, m, Dn)
        Ei = jnp.where(iota_k == kk, sel, Ei)
        Dw = jnp.where(iota_l == sel, jnp.float32(np.inf), Dw)
    eidx_ref[0] = Ei
    idxg_ref[0] = Ei + pl.program_id(0) * N
    rows_k = lax.broadcasted_iota(jnp.int32, (N, K), 0)
    idxd_ref[0] = (Ei + pl.program_id(0) * N
                   + jnp.where(Ei >= rows_k, B * N, 0))

    nA, cA, cC = x[:, 0:3], x[:, 3:6], x[:, 6:9]
    vA = cA - nA
    vB = cC - cA
    nA1 = _shift_up(nA)
    vC = nA1 - cC
    uA = _normalize3(vA)
    uB = _normalize3(vB)
    uC = _normalize3(vC)
    uA1 = _shift_up(uA)
    uB1 = _shift_up(uB)

    def dihed_cs(u2, u1, u0):
        n2 = _normalize3(_cross(u2, u1))
        n1 = _normalize3(_cross(u1, u0))
        cosD = jnp.clip(_dot3(n2, n1), -1.0 + 1e-7, 1.0 - 1e-7)
        sinD = jnp.sign(_dot3(u2, n1)) * jnp.sqrt(1.0 - cosD * cosD)
        return cosD, sinD

    rows = lax.broadcasted_iota(jnp.int32, (N, 1), 0)
    c1, s1 = dihed_cs(uA, uB, uC)
    c2, s2 = dihed_cs(uB, uC, uA1)
    c3, s3 = dihed_cs(uC, uA1, uB1)
    ok = rows <= N - 2
    c1 = jnp.where(ok, c1, 1.0)
    s1 = jnp.where(ok, s1, 0.0)
    c2 = jnp.where(ok, c2, 1.0)
    s2 = jnp.where(ok, s2, 0.0)
    c0 = _shift_down(c3, 1.0)
    s0 = _shift_down(s3, 0.0)
    Vf = jnp.concatenate([c0, c1, c2, s0, s1, s2], 1)

    xca1 = _shift_up(xca)
    Uc = _normalize3(xca1 - xca)
    Uc1 = _shift_up(Uc)
    o1 = _normalize3(Uc - Uc1)
    n2v = _normalize3(_cross(Uc, Uc1))
    o3 = _cross(o1, n2v)
    om9_raw = jnp.concatenate([o1, n2v, o3], 1)
    om9_sh = _shift_down(om9_raw, 0.0)
    okr = jnp.logical_and(rows >= 1, rows <= N - 3)
    Om9 = jnp.where(okr, om9_sh, 0.0)

    TBL = jnp.concatenate([xca, Om9], 1)

    iota8 = lax.broadcasted_iota(jnp.int32, (1, NPE // 2), 1).astype(jnp.float32)
    freq = jnp.exp(iota8 * jnp.float32(-2.0 * np.log(10000.0) / NPE))
    iota16 = lax.broadcasted_iota(jnp.int32, (1, NRBF), 1).astype(jnp.float32)
    mu = iota16 * jnp.float32(20.0 / (NRBF - 1))
    inv_sig = jnp.float32(NRBF / 20.0)
    n_f = rows.astype(jnp.float32)

    for k in range(K):
        sel = Ei[:, k:k + 1]
        dk = sel.astype(jnp.float32) - n_f
        ang = dk * freq
        epos = jnp.concatenate([jnp.cos(ang), jnp.sin(ang)], 1)
        dd = Dn[:, k:k + 1]
        t = (dd - mu) * inv_sig
        rbf = jnp.exp(-t * t)

        oh = (iota_l == sel).astype(jnp.float32)
        Gk = _dot(oh, TBL)
        xn = Gk[:, 0:3]
        on9 = Gk[:, 3:12]
        dXn = xn - xca
        du_cols = []
        for i in range(3):
            acc = (Om9[:, 3 * i:3 * i + 1] * dXn[:, 0:1]
                   + Om9[:, 3 * i + 1:3 * i + 2] * dXn[:, 1:2]
                   + Om9[:, 3 * i + 2:3 * i + 3] * dXn[:, 2:3])
            du_cols.append(acc)
        du = _normalize3(jnp.concatenate(du_cols, 1))
        Rcols = []
        for i in range(3):
            for l in range(3):
                r = (Om9[:, 0 + i:1 + i] * on9[:, 0 + l:1 + l]
                     + Om9[:, 3 + i:4 + i] * on9[:, 3 + l:4 + l]
                     + Om9[:, 6 + i:7 + i] * on9[:, 6 + l:7 + l])
                Rcols.append(r)
        q = _quaternion_cols(Rcols)
        of_k = jnp.concatenate([du, q], 1)

        e_k = jnp.concatenate([epos, rbf, of_k], 1)
        he = _ln_rows(_dot(e_k, edge_w[...]) + edge_b[...], ne_g[...], ne_b[...])
        he_ref[0, :, HID * k:HID * (k + 1)] = _dot(he, w_e[...]) + b_e[...]

    v = _ln_rows(_dot(Vf, node_w[...]) + node_b[...], nn_g[...], nn_b[...])
    hv = _dot(v, w_v[...]) + b_v[...]
    hv_ref[0] = hv
    t_ref[0] = _dot(hv, wkv0[...])


def _layer_body(is_dec, next_kind, C, *refs):
    it = iter(refs)
    hv_ref = next(it)
    he_ref = next(it)
    g_ref = next(it)
    if next_kind == 'dec':
        s_ref = next(it)
        w_s = next(it)
        hve_ref = next(it) if is_dec else None
    wq = next(it)
    wkv_e = next(it)
    m_red = next(it)
    m_exp = next(it)
    m_sum = next(it)
    wo = next(it)
    n0g = next(it)
    n0b = next(it)
    wi = next(it)
    bi = next(it)
    wo2 = next(it)
    bo = next(it)
    n1g = next(it)
    n1b = next(it)
    if next_kind == 'enc':
        wnext = next(it)
    elif next_kind == 'dec':
        wnA = next(it)
        wnB = next(it)
    else:
        wout = next(it)
        bout = next(it)
    hv_out = next(it)
    t_out = next(it)

    hv = hv_ref[0]
    he = he_ref[0]
    g = g_ref[0]

    q = _dot(hv, wq[...])
    scale = jnp.float32(1.0 / np.sqrt(DH))
    kvs = []
    for k in range(K):
        ek = he[:, HID * k:HID * (k + 1)]
        kvs.append(_dot(ek, wkv_e[...]) + g[:, C * k:C * k + 2 * HID])
    kfull = jnp.concatenate([kv[:, :HID] for kv in kvs], 1)
    vfull = jnp.concatenate([kv[:, HID:] for kv in kvs], 1)
    qt = jnp.concatenate([q] * K, 1)
    logits = _dot(qt * kfull, m_red[...]) * scale
    parts = []
    for h in range(NH):
        lg = logits[:, K * h:K * (h + 1)]
        m = jnp.max(lg, 1, keepdims=True)
        e = jnp.exp(lg - m)
        parts.append(e / jnp.sum(e, 1, keepdims=True))
    a_exp = _dot(jnp.concatenate(parts, 1), m_exp[...])
    upd = _dot(_dot(a_exp * vfull, m_sum[...]), wo[...])

    h1 = _ln_rows(hv + upd, n0g[...], n0b[...])
    ffn = _dot(jax.nn.relu(_dot(h1, wi[...]) + bi[...]), wo2[...]) + bo[...]
    h2 = _ln_rows(h1 + ffn, n1g[...], n1b[...])
    hv_out[0] = h2

    if next_kind == 'enc':
        t_out[0] = _dot(h2, wnext[...])
    elif next_kind == 'dec':
        s_col = s_ref[0]
        iota20 = lax.broadcasted_iota(jnp.int32, (1, VOCAB), 1)
        oh_s = (s_col == iota20).astype(jnp.float32)
        hs = _dot(oh_s, w_s[...])
        hve = h2 if not is_dec else hve_ref[0]
        t_out[0] = jnp.concatenate(
            [_dot(hs, wnA[...]) + _dot(h2, wnB[...]), _dot(hve, wnB[...])], 1)
    else:
        lg = _dot(h2, wout[...]) + bout[...]
        m = jnp.max(lg, 1, keepdims=True)
        t_out[0] = lg - m - jnp.log(jnp.sum(jnp.exp(lg - m), 1, keepdims=True))


def _full_spec(shape):
    nd = len(shape)
    return pl.BlockSpec(shape, lambda b, s, _n=nd: (0,) * _n)


def _slab_spec(f):
    return pl.BlockSpec((1, SLAB, f), lambda b, s: (b, s, 0))


def _feature_call(Xr, XcaT, fp, w_v, b_v, w_e, b_e, wkv0):
    in_specs = [pl.BlockSpec((1, N, 12), lambda b: (b, 0, 0)),
                pl.BlockSpec((1, 3, N), lambda b: (b, 0, 0))]
    weights = [fp['node_W'], fp['node_b'].reshape(1, -1), fp['nn_g'].reshape(1, -1),
               fp['nn_b'].reshape(1, -1), fp['edge_W'], fp['edge_b'].reshape(1, -1),
               fp['ne_g'].reshape(1, -1), fp['ne_b'].reshape(1, -1),
               w_v, b_v.reshape(1, -1), w_e, b_e.reshape(1, -1), wkv0]
    for w in weights:
        in_specs.append(pl.BlockSpec(w.shape, lambda b, _n=len(w.shape): (0,) * _n))
    out_shape = [jax.ShapeDtypeStruct((B, N, K * HID), jnp.float32),
                 jax.ShapeDtypeStruct((B, N, HID), jnp.float32),
                 jax.ShapeDtypeStruct((B, N, K), jnp.int32),
                 jax.ShapeDtypeStruct((B, N, K), jnp.int32),
                 jax.ShapeDtypeStruct((B, N, K), jnp.int32),
                 jax.ShapeDtypeStruct((B, N, 2 * HID), jnp.float32)]
    out_specs = [pl.BlockSpec((1, N, K * HID), lambda b: (b, 0, 0)),
                 pl.BlockSpec((1, N, HID), lambda b: (b, 0, 0)),
                 pl.BlockSpec((1, N, K), lambda b: (b, 0, 0)),
                 pl.BlockSpec((1, N, K), lambda b: (b, 0, 0)),
                 pl.BlockSpec((1, N, K), lambda b: (b, 0, 0)),
                 pl.BlockSpec((1, N, 2 * HID), lambda b: (b, 0, 0))]
    return pl.pallas_call(
        _feature_body, grid=(B,), in_specs=in_specs, out_specs=out_specs,
        out_shape=out_shape)(Xr, XcaT, *weights)


def _layer_call(is_dec, next_kind, C, hv, he, g, s_col, w_s, hve, lp,
                next_w):
    args = [hv, he, g]
    in_specs = [_slab_spec(HID), _slab_spec(K * HID), _slab_spec(K * C)]
    if next_kind == 'dec':
        args.append(s_col)
        in_specs.append(_slab_spec(1))
        args.append(w_s)
        in_specs.append(_full_spec(w_s.shape))
        if is_dec:
            args.append(hve)
            in_specs.append(_slab_spec(HID))
    weights = [lp['WQ'], jnp.concatenate([lp['WK'][:HID], lp['WV'][:HID]], 1),
               _M_RED, _M_EXP, _M_SUM,
               lp['WO'], lp['n0_g'].reshape(1, -1), lp['n0_b'].reshape(1, -1),
               lp['Wi'], lp['bi'].reshape(1, -1), lp['Wo'],
               lp['bo'].reshape(1, -1), lp['n1_g'].reshape(1, -1),
               lp['n1_b'].reshape(1, -1)]
    weights += [w for w in next_w]
    for w in weights:
        args.append(w)
        in_specs.append(_full_spec(w.shape))

    out_shape = [jax.ShapeDtypeStruct((B, N, HID), jnp.float32)]
    out_specs = [_slab_spec(HID)]
    if next_kind == 'enc':
        out_shape.append(jax.ShapeDtypeStruct((B, N, 2 * HID), jnp.float32))
        out_specs.append(_slab_spec(2 * HID))
    elif next_kind == 'dec':
        out_shape.append(jax.ShapeDtypeStruct((B, N, 4 * HID), jnp.float32))
        out_specs.append(_slab_spec(4 * HID))
    else:
        out_shape.append(jax.ShapeDtypeStruct((B, N, VOCAB), jnp.float32))
        out_specs.append(_slab_spec(VOCAB))

    body = functools.partial(_layer_body, is_dec, next_kind, C)
    return pl.pallas_call(
        body, grid=(B, NSLAB), in_specs=in_specs, out_specs=out_specs,
        out_shape=out_shape)(*args)


def _sc_gather(table, idx, C):
    M = idx.shape[0]
    NW = 32
    per_w = M // NW
    chunk = 384 if C <= 256 else 192
    n_it = per_w // chunk
    mesh = plsc.VectorSubcoreMesh(core_axis_name="c", subcore_axis_name="s")

    def body(table_ref, idx_ref, out_ref, idx_v, rows_v, sem):
        wid = lax.axis_index("s") * 2 + lax.axis_index("c")
        base = wid * per_w

        def it(i, carry):
            off = base + i * chunk
            pltpu.sync_copy(idx_ref.at[pl.ds(off, chunk)], idx_v)
            pltpu.async_copy(table_ref.at[idx_v], rows_v, sem).wait()
            pltpu.sync_copy(rows_v, out_ref.at[pl.ds(off, chunk)])
            return carry

        lax.fori_loop(0, n_it, it, 0)

    f = pl.kernel(body,
                  out_type=jax.ShapeDtypeStruct((M, C), jnp.float32),
                  mesh=mesh,
                  scratch_types=[pltpu.VMEM((chunk,), jnp.int32),
                                 pltpu.VMEM((chunk, C), jnp.float32),
                                 pltpu.SemaphoreType.DMA])
    return f(table, idx)


_gather = _sc_gather


def kernel(X, S, L, mask, params):
    fp = params['feat']
    enc = params['enc']
    dec = params['dec']
    Xr = X.reshape(B, N, 12)
    XcaT = jnp.transpose(X[:, :, 1, :], (0, 2, 1))
    s_col = S.reshape(B, N, 1).astype(jnp.int32)

    def enc_tbl_w(lp):
        return jnp.concatenate([lp['WK'][HID:], lp['WV'][HID:]], 1)

    def dec_tbl_w(lp):
        wnA = jnp.concatenate([lp['WK'][HID:2 * HID], lp['WV'][HID:2 * HID]], 1)
        wnB = jnp.concatenate([lp['WK'][2 * HID:], lp['WV'][2 * HID:]], 1)
        return wnA, wnB

    hE, hV, eidx, idxg, idxd, T = _feature_call(
        Xr, XcaT, fp, params['W_v'], params['b_v'], params['W_e'],
        params['b_e'], enc_tbl_w(enc[0]))
    idx_flat = idxg.reshape(B * N * K)
    idxd_flat = idxd.reshape(B * N * K)

    def dec_table(T4):
        return jnp.concatenate([T4[:, :, :2 * HID].reshape(B * N, 2 * HID),
                                T4[:, :, 2 * HID:].reshape(B * N, 2 * HID)], 0)

    for i in range(3):
        lp = enc[i]
        G = _gather(T.reshape(B * N, 2 * HID), idx_flat, 2 * HID)
        G = G.reshape(B, N, K * 2 * HID)
        if i < 2:
            nk, nw = 'enc', (enc_tbl_w(enc[i + 1]),)
        else:
            nk, nw = 'dec', dec_tbl_w(dec[0])
        hV, T = _layer_call(False, nk, 2 * HID, hV, hE, G,
                            s_col if nk == 'dec' else None,
                            params['W_s'] if nk == 'dec' else None,
                            None, lp, nw)

    hVe = hV
    for i in range(3):
        lp = dec[i]
        G = _gather(dec_table(T), idxd_flat, 2 * HID)
        G = G.reshape(B, N, K * 2 * HID)
        if i < 2:
            nk, nw = 'dec', dec_tbl_w(dec[i + 1])
        else:
            nk, nw = None, (params['W_out'], params['b_out'].reshape(1, -1))
        hV, T = _layer_call(True, nk, 2 * HID, hV, hE, G,
                            s_col if nk == 'dec' else None,
                            params['W_s'] if nk == 'dec' else None,
                            hVe if nk == 'dec' else None, lp, nw)
    return T


# speedup vs baseline: 4.9963x; 1.4882x over previous
import functools
import numpy as np
import jax
import jax.numpy as jnp
from jax import lax
from jax.experimental import pallas as pl
from jax.experimental.pallas import tpu as pltpu
from jax.experimental.pallas import tpu_sc as plsc

B, N, K = 4, 512, 30
HID = 128
NH, DH = 4, 32
NPE, NRBF = 16, 16
VOCAB = 20
SLAB = 256
NSLAB = N // SLAB

_PREC = lax.Precision.HIGHEST


def _attn_mats():
    j = np.arange(K * HID)
    r = j % HID
    mcol = (r // DH) * K + (j // HID)
    m_red = np.zeros((K * HID, NH * K), np.float32)
    m_red[j, mcol] = 1.0
    m_sum = np.zeros((K * HID, HID), np.float32)
    m_sum[j, r] = 1.0
    return (jnp.asarray(m_red), jnp.asarray(np.ascontiguousarray(m_red.T)),
            jnp.asarray(m_sum))


_M_RED, _M_EXP, _M_SUM = _attn_mats()


def _dot(a, b):
    return lax.dot_general(a, b, (((1,), (0,)), ((), ())), precision=_PREC,
                           preferred_element_type=jnp.float32)


def _dotf(a, b):
    return lax.dot_general(a, b, (((1,), (0,)), ((), ())),
                           preferred_element_type=jnp.float32)


def _ln_rows(x, g, b):
    n = x.shape[-1]
    mu = jnp.mean(x, -1, keepdims=True)
    d = x - mu
    var = jnp.sum(d * d, -1, keepdims=True) / (n - 1)
    sigma = jnp.sqrt(var + 1e-6)
    return g * d / (sigma + 1e-6) + b


def _normalize3(v, eps=1e-12):
    n = jnp.sqrt(jnp.sum(v * v, -1, keepdims=True))
    return v / jnp.maximum(n, eps)


def _cross(a, b):
    ax, ay, az = a[:, 0:1], a[:, 1:2], a[:, 2:3]
    bx, by, bz = b[:, 0:1], b[:, 1:2], b[:, 2:3]
    return jnp.concatenate([ay * bz - az * by, az * bx - ax * bz,
                            ax * by - ay * bx], 1)


def _dot3(a, b):
    return jnp.sum(a * b, -1, keepdims=True)


def _shift_up(z):
    return jnp.concatenate([z[1:], jnp.zeros((1, z.shape[1]), z.dtype)], 0)


def _shift_down(z, fill=0.0):
    return jnp.concatenate([jnp.full((1, z.shape[1]), fill, z.dtype), z[:-1]], 0)


def _quaternion_cols(R):
    Rxx, Ryy, Rzz = R[0], R[4], R[8]
    m1 = 0.5 * jnp.sqrt(jnp.abs(1.0 + Rxx - Ryy - Rzz) + 1e-10)
    m2 = 0.5 * jnp.sqrt(jnp.abs(1.0 - Rxx + Ryy - Rzz) + 1e-10)
    m3 = 0.5 * jnp.sqrt(jnp.abs(1.0 - Rxx - Ryy + Rzz) + 1e-10)
    s1 = jnp.sign(R[7] - R[5])
    s2 = jnp.sign(R[2] - R[6])
    s3 = jnp.sign(R[3] - R[1])
    w = jnp.sqrt(jax.nn.relu(1.0 + Rxx + Ryy + Rzz) + 1e-10) / 2.0
    q = jnp.concatenate([s1 * m1, s2 * m2, s3 * m3, w], 1)
    return _normalize3(q)


def _feature_body(x_ref, xcat_ref, node_w, node_b, nn_g, nn_b, edge_w, edge_b,
                  ne_g, ne_b, w_v, b_v, w_e, b_e, wkv0,
                  he_ref, hv_ref, eidx_ref, idxg_ref, idxd_ref, t_ref):
    x = x_ref[0]
    xcat = xcat_ref[0]
    xca = x[:, 3:6]

    D2 = jnp.zeros((N, N), jnp.float32)
    for c in range(3):
        dc = xca[:, c:c + 1] - xcat[c:c + 1, :]
        D2 = D2 + dc * dc
    D = jnp.sqrt(D2 + 1e-6)

    iota_l = lax.broadcasted_iota(jnp.int32, (N, N), 1)
    iota_k = lax.broadcasted_iota(jnp.int32, (N, K), 1)
    Dw = D
    Dn = jnp.zeros((N, K), jnp.float32)
    Ei = jnp.zeros((N, K), jnp.int32)
    for kk in range(K):
        m = jnp.min(Dw, axis=1, keepdims=True)
        sel = jnp.min(jnp.where(Dw == m, iota_l, N), axis=1, keepdims=True)
        Dn = jnp.where(iota_k == kk, m, Dn)
        Ei = jnp.where(iota_k == kk, sel, Ei)
        Dw = jnp.where(iota_l == sel, jnp.float32(np.inf), Dw)
    eidx_ref[0] = Ei
    idxg_ref[0] = Ei + pl.program_id(0) * N
    rows_k = lax.broadcasted_iota(jnp.int32, (N, K), 0)
    idxd_ref[0] = (Ei + pl.program_id(0) * N
                   + jnp.where(Ei >= rows_k, B * N, 0))

    nA, cA, cC = x[:, 0:3], x[:, 3:6], x[:, 6:9]
    vA = cA - nA
    vB = cC - cA
    nA1 = _shift_up(nA)
    vC = nA1 - cC
    uA = _normalize3(vA)
    uB = _normalize3(vB)
    uC = _normalize3(vC)
    uA1 = _shift_up(uA)
    uB1 = _shift_up(uB)

    def dihed_cs(u2, u1, u0):
        n2 = _normalize3(_cross(u2, u1))
        n1 = _normalize3(_cross(u1, u0))
        cosD = jnp.clip(_dot3(n2, n1), -1.0 + 1e-7, 1.0 - 1e-7)
        sinD = jnp.sign(_dot3(u2, n1)) * jnp.sqrt(1.0 - cosD * cosD)
        return cosD, sinD

    rows = lax.broadcasted_iota(jnp.int32, (N, 1), 0)
    c1, s1 = dihed_cs(uA, uB, uC)
    c2, s2 = dihed_cs(uB, uC, uA1)
    c3, s3 = dihed_cs(uC, uA1, uB1)
    ok = rows <= N - 2
    c1 = jnp.where(ok, c1, 1.0)
    s1 = jnp.where(ok, s1, 0.0)
    c2 = jnp.where(ok, c2, 1.0)
    s2 = jnp.where(ok, s2, 0.0)
    c0 = _shift_down(c3, 1.0)
    s0 = _shift_down(s3, 0.0)
    Vf = jnp.concatenate([c0, c1, c2, s0, s1, s2], 1)

    xca1 = _shift_up(xca)
    Uc = _normalize3(xca1 - xca)
    Uc1 = _shift_up(Uc)
    o1 = _normalize3(Uc - Uc1)
    n2v = _normalize3(_cross(Uc, Uc1))
    o3 = _cross(o1, n2v)
    om9_raw = jnp.concatenate([o1, n2v, o3], 1)
    om9_sh = _shift_down(om9_raw, 0.0)
    okr = jnp.logical_and(rows >= 1, rows <= N - 3)
    Om9 = jnp.where(okr, om9_sh, 0.0)

    TBL = jnp.concatenate([xca, Om9], 1)

    iota8 = lax.broadcasted_iota(jnp.int32, (1, NPE // 2), 1).astype(jnp.float32)
    freq = jnp.exp(iota8 * jnp.float32(-2.0 * np.log(10000.0) / NPE))
    iota16 = lax.broadcasted_iota(jnp.int32, (1, NRBF), 1).astype(jnp.float32)
    mu = iota16 * jnp.float32(20.0 / (NRBF - 1))
    inv_sig = jnp.float32(NRBF / 20.0)
    n_f = rows.astype(jnp.float32)

    for k in range(K):
        sel = Ei[:, k:k + 1]
        dk = sel.astype(jnp.float32) - n_f
        ang = dk * freq
        epos = jnp.concatenate([jnp.cos(ang), jnp.sin(ang)], 1)
        dd = Dn[:, k:k + 1]
        t = (dd - mu) * inv_sig
        rbf = jnp.exp(-t * t)

        oh = (iota_l == sel).astype(jnp.float32)
        Gk = _dot(oh, TBL)
        xn = Gk[:, 0:3]
        on9 = Gk[:, 3:12]
        dXn = xn - xca
        du_cols = []
        for i in range(3):
            acc = (Om9[:, 3 * i:3 * i + 1] * dXn[:, 0:1]
                   + Om9[:, 3 * i + 1:3 * i + 2] * dXn[:, 1:2]
                   + Om9[:, 3 * i + 2:3 * i + 3] * dXn[:, 2:3])
            du_cols.append(acc)
        du = _normalize3(jnp.concatenate(du_cols, 1))
        Rcols = []
        for i in range(3):
            for l in range(3):
                r = (Om9[:, 0 + i:1 + i] * on9[:, 0 + l:1 + l]
                     + Om9[:, 3 + i:4 + i] * on9[:, 3 + l:4 + l]
                     + Om9[:, 6 + i:7 + i] * on9[:, 6 + l:7 + l])
                Rcols.append(r)
        q = _quaternion_cols(Rcols)
        of_k = jnp.concatenate([du, q], 1)

        e_k = jnp.concatenate([epos, rbf, of_k], 1)
        he = _ln_rows(_dotf(e_k, edge_w[...]) + edge_b[...], ne_g[...], ne_b[...])
        he_ref[0, :, HID * k:HID * (k + 1)] = _dotf(he, w_e[...]) + b_e[...]

    v = _ln_rows(_dotf(Vf, node_w[...]) + node_b[...], nn_g[...], nn_b[...])
    hv = _dotf(v, w_v[...]) + b_v[...]
    hv_ref[0] = hv
    t_ref[0] = _dotf(hv, wkv0[...])


def _layer_body(is_dec, next_kind, C, *refs):
    it = iter(refs)
    hv_ref = next(it)
    he_ref = next(it)
    g_ref = next(it)
    if next_kind == 'dec':
        s_ref = next(it)
        w_s = next(it)
        hve_ref = next(it) if is_dec else None
    wq = next(it)
    wkv_e = next(it)
    m_red = next(it)
    m_exp = next(it)
    m_sum = next(it)
    wo = next(it)
    n0g = next(it)
    n0b = next(it)
    wi = next(it)
    bi = next(it)
    wo2 = next(it)
    bo = next(it)
    n1g = next(it)
    n1b = next(it)
    if next_kind == 'enc':
        wnext = next(it)
    elif next_kind == 'dec':
        wnA = next(it)
        wnB = next(it)
    else:
        wout = next(it)
        bout = next(it)
    hv_out = next(it)
    t_out = next(it)

    hv = hv_ref[0]
    he = he_ref[0]
    g = g_ref[0]

    q = _dotf(hv, wq[...])
    scale = jnp.float32(1.0 / np.sqrt(DH))
    kvs = []
    for k in range(K):
        ek = he[:, HID * k:HID * (k + 1)]
        kvs.append(_dotf(ek, wkv_e[...]) + g[:, C * k:C * k + 2 * HID])
    kfull = jnp.concatenate([kv[:, :HID] for kv in kvs], 1)
    vfull = jnp.concatenate([kv[:, HID:] for kv in kvs], 1)
    qt = jnp.concatenate([q] * K, 1)
    logits = _dotf(qt * kfull, m_red[...]) * scale
    parts = []
    for h in range(NH):
        lg = logits[:, K * h:K * (h + 1)]
        m = jnp.max(lg, 1, keepdims=True)
        e = jnp.exp(lg - m)
        parts.append(e / jnp.sum(e, 1, keepdims=True))
    a_exp = _dotf(jnp.concatenate(parts, 1), m_exp[...])
    upd = _dotf(_dotf(a_exp * vfull, m_sum[...]), wo[...])

    h1 = _ln_rows(hv + upd, n0g[...], n0b[...])
    ffn = _dotf(jax.nn.relu(_dotf(h1, wi[...]) + bi[...]), wo2[...]) + bo[...]
    h2 = _ln_rows(h1 + ffn, n1g[...], n1b[...])
    hv_out[0] = h2

    if next_kind == 'enc':
        t_out[0] = _dotf(h2, wnext[...])
    elif next_kind == 'dec':
        s_col = s_ref[0]
        iota20 = lax.broadcasted_iota(jnp.int32, (1, VOCAB), 1)
        oh_s = (s_col == iota20).astype(jnp.float32)
        hs = _dot(oh_s, w_s[...])
        hve = h2 if not is_dec else hve_ref[0]
        t_out[0] = jnp.concatenate(
            [_dotf(hs, wnA[...]) + _dotf(h2, wnB[...]), _dotf(hve, wnB[...])], 1)
    else:
        lg = _dotf(h2, wout[...]) + bout[...]
        m = jnp.max(lg, 1, keepdims=True)
        t_out[0] = lg - m - jnp.log(jnp.sum(jnp.exp(lg - m), 1, keepdims=True))


def _full_spec(shape):
    nd = len(shape)
    return pl.BlockSpec(shape, lambda b, s, _n=nd: (0,) * _n)


def _slab_spec(f):
    return pl.BlockSpec((1, SLAB, f), lambda b, s: (b, s, 0))


def _feature_call(Xr, XcaT, fp, w_v, b_v, w_e, b_e, wkv0):
    in_specs = [pl.BlockSpec((1, N, 12), lambda b: (b, 0, 0)),
                pl.BlockSpec((1, 3, N), lambda b: (b, 0, 0))]
    weights = [fp['node_W'], fp['node_b'].reshape(1, -1), fp['nn_g'].reshape(1, -1),
               fp['nn_b'].reshape(1, -1), fp['edge_W'], fp['edge_b'].reshape(1, -1),
               fp['ne_g'].reshape(1, -1), fp['ne_b'].reshape(1, -1),
               w_v, b_v.reshape(1, -1), w_e, b_e.reshape(1, -1), wkv0]
    for w in weights:
        in_specs.append(pl.BlockSpec(w.shape, lambda b, _n=len(w.shape): (0,) * _n))
    out_shape = [jax.ShapeDtypeStruct((B, N, K * HID), jnp.float32),
                 jax.ShapeDtypeStruct((B, N, HID), jnp.float32),
                 jax.ShapeDtypeStruct((B, N, K), jnp.int32),
                 jax.ShapeDtypeStruct((B, N, K), jnp.int32),
                 jax.ShapeDtypeStruct((B, N, K), jnp.int32),
                 jax.ShapeDtypeStruct((B, N, 2 * HID), jnp.float32)]
    out_specs = [pl.BlockSpec((1, N, K * HID), lambda b: (b, 0, 0)),
                 pl.BlockSpec((1, N, HID), lambda b: (b, 0, 0)),
                 pl.BlockSpec((1, N, K), lambda b: (b, 0, 0)),
                 pl.BlockSpec((1, N, K), lambda b: (b, 0, 0)),
                 pl.BlockSpec((1, N, K), lambda b: (b, 0, 0)),
                 pl.BlockSpec((1, N, 2 * HID), lambda b: (b, 0, 0))]
    return pl.pallas_call(
        _feature_body, grid=(B,), in_specs=in_specs, out_specs=out_specs,
        out_shape=out_shape)(Xr, XcaT, *weights)


def _layer_call(is_dec, next_kind, C, hv, he, g, s_col, w_s, hve, lp,
                next_w):
    args = [hv, he, g]
    in_specs = [_slab_spec(HID), _slab_spec(K * HID), _slab_spec(K * C)]
    if next_kind == 'dec':
        args.append(s_col)
        in_specs.append(_slab_spec(1))
        args.append(w_s)
        in_specs.append(_full_spec(w_s.shape))
        if is_dec:
            args.append(hve)
            in_specs.append(_slab_spec(HID))
    weights = [lp['WQ'], jnp.concatenate([lp['WK'][:HID], lp['WV'][:HID]], 1),
               _M_RED, _M_EXP, _M_SUM,
               lp['WO'], lp['n0_g'].reshape(1, -1), lp['n0_b'].reshape(1, -1),
               lp['Wi'], lp['bi'].reshape(1, -1), lp['Wo'],
               lp['bo'].reshape(1, -1), lp['n1_g'].reshape(1, -1),
               lp['n1_b'].reshape(1, -1)]
    weights += [w for w in next_w]
    for w in weights:
        args.append(w)
        in_specs.append(_full_spec(w.shape))

    out_shape = [jax.ShapeDtypeStruct((B, N, HID), jnp.float32)]
    out_specs = [_slab_spec(HID)]
    if next_kind == 'enc':
        out_shape.append(jax.ShapeDtypeStruct((B, N, 2 * HID), jnp.float32))
        out_specs.append(_slab_spec(2 * HID))
    elif next_kind == 'dec':
        out_shape.append(jax.ShapeDtypeStruct((B, N, 4 * HID), jnp.float32))
        out_specs.append(_slab_spec(4 * HID))
    else:
        out_shape.append(jax.ShapeDtypeStruct((B, N, VOCAB), jnp.float32))
        out_specs.append(_slab_spec(VOCAB))

    body = functools.partial(_layer_body, is_dec, next_kind, C)
    return pl.pallas_call(
        body, grid=(B, NSLAB), in_specs=in_specs, out_specs=out_specs,
        out_shape=out_shape)(*args)


def _sc_gather(table, idx, C):
    M = idx.shape[0]
    NW = 32
    per_w = M // NW
    chunk = 384 if C <= 256 else 192
    n_it = per_w // chunk
    mesh = plsc.VectorSubcoreMesh(core_axis_name="c", subcore_axis_name="s")

    def body(table_ref, idx_ref, out_ref, idx_v, rows_v, sem):
        wid = lax.axis_index("s") * 2 + lax.axis_index("c")
        base = wid * per_w

        def it(i, carry):
            off = base + i * chunk
            pltpu.sync_copy(idx_ref.at[pl.ds(off, chunk)], idx_v)
            pltpu.async_copy(table_ref.at[idx_v], rows_v, sem).wait()
            pltpu.sync_copy(rows_v, out_ref.at[pl.ds(off, chunk)])
            return carry

        lax.fori_loop(0, n_it, it, 0)

    f = pl.kernel(body,
                  out_type=jax.ShapeDtypeStruct((M, C), jnp.float32),
                  mesh=mesh,
                  scratch_types=[pltpu.VMEM((chunk,), jnp.int32),
                                 pltpu.VMEM((chunk, C), jnp.float32),
                                 pltpu.SemaphoreType.DMA])
    return f(table, idx)


_gather = _sc_gather


def kernel(X, S, L, mask, params):
    fp = params['feat']
    enc = params['enc']
    dec = params['dec']
    Xr = X.reshape(B, N, 12)
    XcaT = jnp.transpose(X[:, :, 1, :], (0, 2, 1))
    s_col = S.reshape(B, N, 1).astype(jnp.int32)

    def enc_tbl_w(lp):
        return jnp.concatenate([lp['WK'][HID:], lp['WV'][HID:]], 1)

    def dec_tbl_w(lp):
        wnA = jnp.concatenate([lp['WK'][HID:2 * HID], lp['WV'][HID:2 * HID]], 1)
        wnB = jnp.concatenate([lp['WK'][2 * HID:], lp['WV'][2 * HID:]], 1)
        return wnA, wnB

    hE, hV, eidx, idxg, idxd, T = _feature_call(
        Xr, XcaT, fp, params['W_v'], params['b_v'], params['W_e'],
        params['b_e'], enc_tbl_w(enc[0]))
    idx_flat = idxg.reshape(B * N * K)
    idxd_flat = idxd.reshape(B * N * K)

    def dec_table(T4):
        return jnp.concatenate([T4[:, :, :2 * HID].reshape(B * N, 2 * HID),
                                T4[:, :, 2 * HID:].reshape(B * N, 2 * HID)], 0)

    for i in range(3):
        lp = enc[i]
        G = _gather(T.reshape(B * N, 2 * HID), idx_flat, 2 * HID)
        G = G.reshape(B, N, K * 2 * HID)
        if i < 2:
            nk, nw = 'enc', (enc_tbl_w(enc[i + 1]),)
        else:
            nk, nw = 'dec', dec_tbl_w(dec[0])
        hV, T = _layer_call(False, nk, 2 * HID, hV, hE, G,
                            s_col if nk == 'dec' else None,
                            params['W_s'] if nk == 'dec' else None,
                            None, lp, nw)

    hVe = hV
    for i in range(3):
        lp = dec[i]
        G = _gather(dec_table(T), idxd_flat, 2 * HID)
        G = G.reshape(B, N, K * 2 * HID)
        if i < 2:
            nk, nw = 'dec', dec_tbl_w(dec[i + 1])
        else:
            nk, nw = None, (params['W_out'], params['b_out'].reshape(1, -1))
        hV, T = _layer_call(True, nk, 2 * HID, hV, hE, G,
                            s_col if nk == 'dec' else None,
                            params['W_s'] if nk == 'dec' else None,
                            hVe if nk == 'dec' else None, lp, nw)
    return T
